# Initial kernel scaffold; baseline (speedup 1.0000x reference)
import jax
import jax.numpy as jnp
from jax.experimental import pallas as pl


def kernel(x, edge_index, edge_attr, return_attention_weights, W_l, W_r, W_e, att, bias):
    raise NotImplementedError("write your pallas kernel here")


# speedup vs baseline: 2.1230x; 2.1230x over previous
import functools

import jax
import jax.numpy as jnp
from jax import lax
from jax.experimental import pallas as pl
from jax.experimental.pallas import tpu as pltpu
from jax.experimental.pallas import tpu_sc as plsc

N = 10000
E = 160000
D = 256
DH = 128
DE = 16
NEG = 0.2
NEG_INF = -3e38
NP = 12800
NBLK = 512

NT2 = 32
EPT2 = E // NT2
K2B = 40
NB2 = EPT2 // K2B

EPT3 = E // 16
K3B = 40
NB3 = EPT3 // K3B


def _kee_body(ea_ref, we_ref, ee_ref, sum_ref):
    i = pl.program_id(0)
    ea = ea_ref[...]
    ee_ref[...] = jnp.dot(ea, we_ref[...], preferred_element_type=jnp.float32)

    @pl.when(i == 0)
    def _():
        sum_ref[...] = jnp.zeros((1, DE), jnp.float32)

    sum_ref[...] += jnp.sum(ea, axis=0, keepdims=True)


def _kee(edge_attr, W_e):
    blk = 2000
    return pl.pallas_call(
        _kee_body,
        grid=(E // blk,),
        in_specs=[
            pl.BlockSpec((blk, DE), lambda i: (i, 0)),
            pl.BlockSpec((DE, D), lambda i: (0, 0)),
        ],
        out_specs=[
            pl.BlockSpec((blk, D), lambda i: (i, 0)),
            pl.BlockSpec((1, DE), lambda i: (0, 0)),
        ],
        out_shape=[
            jax.ShapeDtypeStruct((E, D), jnp.float32),
            jax.ShapeDtypeStruct((1, DE), jnp.float32),
        ],
    )(edge_attr, W_e)


def _k1_body(x_ref, wl_ref, wr_ref, we_ref, sum_ref, att_ref,
             xl0_ref, xl1_ref, xr0_ref, xr1_ref, m_ref):
    i = pl.program_id(0)
    xv = x_ref[...]
    xl = jnp.dot(xv, wl_ref[...], preferred_element_type=jnp.float32)
    xr = jnp.dot(xv, wr_ref[...], preferred_element_type=jnp.float32)
    xl0_ref[...] = xl[:, :DH]
    xl1_ref[...] = xl[:, DH:]
    xr0_ref[...] = xr[:, :DH]
    xr1_ref[...] = xr[:, DH:]
    crow = jnp.dot(sum_ref[...] * (1.0 / E), we_ref[...],
                   preferred_element_type=jnp.float32)
    z = xl + xr + crow
    z = jnp.maximum(z, NEG * z)
    ls = jnp.sum(z * att_ref[...], axis=1)
    m = jnp.max(ls)

    @pl.when(i == 0)
    def _():
        m_ref[...] = jnp.full((8, 128), NEG_INF, jnp.float32)

    m_ref[...] = jnp.maximum(m_ref[...], m)


def _k1(xp, W_l, W_r, W_e, sumea, att2):
    return pl.pallas_call(
        _k1_body,
        grid=(NP // NBLK,),
        in_specs=[
            pl.BlockSpec((NBLK, D), lambda i: (i, 0)),
            pl.BlockSpec((D, D), lambda i: (0, 0)),
            pl.BlockSpec((D, D), lambda i: (0, 0)),
            pl.BlockSpec((DE, D), lambda i: (0, 0)),
            pl.BlockSpec((1, DE), lambda i: (0, 0)),
            pl.BlockSpec((1, D), lambda i: (0, 0)),
        ],
        out_specs=[
            pl.BlockSpec((NBLK, DH), lambda i: (i, 0)),
            pl.BlockSpec((NBLK, DH), lambda i: (i, 0)),
            pl.BlockSpec((NBLK, DH), lambda i: (i, 0)),
            pl.BlockSpec((NBLK, DH), lambda i: (i, 0)),
            pl.BlockSpec((8, 128), lambda i: (0, 0)),
        ],
        out_shape=[
            jax.ShapeDtypeStruct((NP, DH), jnp.float32),
            jax.ShapeDtypeStruct((NP, DH), jnp.float32),
            jax.ShapeDtypeStruct((NP, DH), jnp.float32),
            jax.ShapeDtypeStruct((NP, DH), jnp.float32),
            jax.ShapeDtypeStruct((8, 128), jnp.float32),
        ],
    )(xp, W_l, W_r, W_e, sumea, att2)


def _k2_body(xl0, xl1, xr0, xr1, ee, esrc, edst, att,
             logits, tmax,
             srcb, dstb, bxl0, bxl1, bxr0, bxr1, bee, lbuf, attv, mxb, tbuf):
    c = lax.axis_index("c")
    s = lax.axis_index("s")
    wid = s * 2 + c
    base = wid * EPT2

    pltpu.sync_copy(att, attv)
    attc = [attv[pl.ds(16 * k, 16)] for k in range(16)]
    iota = lax.broadcasted_iota(jnp.int32, (16,), 0)

    def blk(g, mx16):
        e0 = base + g * K2B
        pltpu.sync_copy(esrc.at[pl.ds(e0, K2B)], srcb)
        pltpu.sync_copy(edst.at[pl.ds(e0, K2B)], dstb)
        pltpu.sync_copy(xl0.at[srcb], bxl0)
        pltpu.sync_copy(xl1.at[srcb], bxl1)
        pltpu.sync_copy(xr0.at[dstb], bxr0)
        pltpu.sync_copy(xr1.at[dstb], bxr1)
        pltpu.sync_copy(ee.at[pl.ds(e0, K2B)], bee)

        for gg in range(3):
            ne = 16 if gg < 2 else K2B - 32

            def edge(je, _):
                j = gg * 16 + je
                t = jnp.zeros((16,), jnp.float32)
                for k in range(8):
                    z = (bxl0[j, pl.ds(16 * k, 16)]
                         + bxr0[j, pl.ds(16 * k, 16)]
                         + bee[j, pl.ds(16 * k, 16)])
                    z = jnp.maximum(z, NEG * z)
                    t = t + attc[k] * z
                for k in range(8):
                    z = (bxl1[j, pl.ds(16 * k, 16)]
                         + bxr1[j, pl.ds(16 * k, 16)]
                         + bee[j, pl.ds(DH + 16 * k, 16)])
                    z = jnp.maximum(z, NEG * z)
                    t = t + attc[8 + k] * z
                plsc.store_scatter(tbuf, [iota * 16 + je], t)
                return 0

            lax.fori_loop(0, ne, edge, 0)
            colsum = tbuf[pl.ds(0, 16)]
            for r in range(1, 16):
                colsum = colsum + tbuf[pl.ds(16 * r, 16)]
            lbuf[pl.ds(16 * gg, 16)] = colsum
            mx16 = jnp.maximum(mx16, colsum)
        pltpu.sync_copy(lbuf.at[pl.ds(0, K2B)], logits.at[pl.ds(e0, K2B)])
        return mx16

    mx16 = lax.fori_loop(0, NB2, blk, jnp.full((16,), NEG_INF, jnp.float32))
    mxb[...] = mx16
    pltpu.sync_copy(mxb, tmax.at[wid])


def _k2(xl0, xl1, xr0, xr1, ee, esrc, edst, att):
    mesh = plsc.VectorSubcoreMesh(core_axis_name="c", subcore_axis_name="s")
    f = functools.partial(
        pl.kernel,
        out_type=[
            jax.ShapeDtypeStruct((E,), jnp.float32),
            jax.ShapeDtypeStruct((NT2, 16), jnp.float32),
        ],
        mesh=mesh,
        scratch_types=[
            pltpu.VMEM((K2B,), jnp.int32),
            pltpu.VMEM((K2B,), jnp.int32),
            pltpu.VMEM((K2B, DH), jnp.float32),
            pltpu.VMEM((K2B, DH), jnp.float32),
            pltpu.VMEM((K2B, DH), jnp.float32),
            pltpu.VMEM((K2B, DH), jnp.float32),
            pltpu.VMEM((K2B, D), jnp.float32),
            pltpu.VMEM((48,), jnp.float32),
            pltpu.VMEM((D,), jnp.float32),
            pltpu.VMEM((16,), jnp.float32),
            pltpu.VMEM((D,), jnp.float32),
        ],
        compiler_params=pltpu.CompilerParams(needs_layout_passes=False),
    )(_k2_body)
    return f(xl0, xl1, xr0, xr1, ee, esrc, edst, att)


def _k3_body(xl0, xl1, esrc, edst, logits, tmax, mself,
             accout, sparts,
             acc, srcb, dstb, gb, stage, lb, exb, tb, msb, stab):
    c = lax.axis_index("c")
    s = lax.axis_index("s")
    iota = lax.broadcasted_iota(jnp.int32, (16,), 0)

    pltpu.sync_copy(tmax, tb)
    pltpu.sync_copy(mself.at[0, pl.ds(0, 16)], msb)
    m16 = msb[...]
    for r in range(NT2):
        m16 = jnp.maximum(m16, tb[r])
    for sh in (8, 4, 2, 1):
        msb[...] = m16
        m16 = jnp.maximum(m16, plsc.load_gather(msb, [(iota + sh) & 15]))

    z16 = jnp.zeros((16,), jnp.float32)

    def zrow(i, _):
        for k in range(DH // 16):
            stage[i, pl.ds(16 * k, 16)] = z16
        return 0

    lax.fori_loop(0, K3B, zrow, 0)

    def zacc(q, _):
        pltpu.sync_copy(stage, acc.at[pl.ds(s * 800 + q * K3B, K3B)])
        return 0

    lax.fori_loop(0, 800 // K3B, zacc, 0)

    @pl.when(c == 0)
    def _():
        def zs(i, _):
            stab[pl.ds(16 * i, 16)] = z16
            return 0

        lax.fori_loop(0, NP // 16, zs, 0)

    plsc.subcore_barrier()

    ebase = s * EPT3

    def main(xtab, do_s):
        def blk(g, _):
            e0 = ebase + g * K3B
            pltpu.sync_copy(esrc.at[pl.ds(e0, K3B)], srcb)
            pltpu.sync_copy(edst.at[pl.ds(e0, K3B)], dstb)
            pltpu.sync_copy(xtab.at[srcb], gb)
            pltpu.sync_copy(logits.at[pl.ds(e0, K3B)], lb.at[pl.ds(0, K3B)])
            for grp in range(3):
                lv = lb[pl.ds(16 * grp, 16)]
                exb[pl.ds(16 * grp, 16)] = jnp.exp(lv - m16)

            if do_s:
                for off, lo in ((0, 0), (16, 0), (24, 8)):
                    dst16 = dstb[pl.ds(off, 16)]
                    ex16 = exb[pl.ds(off, 16)]
                    for r in range(lo, 16):
                        plsc.addupdate_scatter(stab, [dst16], ex16,
                                               mask=iota == r)

            def edge(j, _):
                exs = plsc.load_gather(exb, [jnp.full((16,), j, jnp.int32)])
                for k in range(DH // 16):
                    stage[j, pl.ds(16 * k, 16)] = gb[j, pl.ds(16 * k, 16)] * exs
                return 0

            lax.fori_loop(0, K3B, edge, 0)
            pltpu.sync_copy(stage, acc.at[dstb], add=True)
            return 0

        lax.fori_loop(0, NB3, blk, 0)

    @pl.when(c == 0)
    def _():
        main(xl0, True)

    @pl.when(c == 1)
    def _():
        main(xl1, False)

    plsc.subcore_barrier()
    pltpu.sync_copy(acc.at[pl.ds(s * 800, 800)],
                    accout.at[pl.ds(c * NP + s * 800, 800)])

    @pl.when(c == 0)
    def _():
        pltpu.sync_copy(stab, sparts.at[pl.ds(s * NP, NP)])


def _k3(xl0, xl1, esrc, edst, logits, tmax, mself):
    mesh = plsc.VectorSubcoreMesh(core_axis_name="c", subcore_axis_name="s")
    f = functools.partial(
        pl.kernel,
        out_type=[
            jax.ShapeDtypeStruct((2 * NP, DH), jnp.float32),
            jax.ShapeDtypeStruct((16 * NP,), jnp.float32),
        ],
        mesh=mesh,
        scratch_types=[
            pltpu.VMEM_SHARED((NP, DH), jnp.float32),
            pltpu.VMEM((K3B,), jnp.int32),
            pltpu.VMEM((K3B,), jnp.int32),
            pltpu.VMEM((K3B, DH), jnp.float32),
            pltpu.VMEM((K3B, DH), jnp.float32),
            pltpu.VMEM((48,), jnp.float32),
            pltpu.VMEM((48,), jnp.float32),
            pltpu.VMEM((NT2, 16), jnp.float32),
            pltpu.VMEM((16,), jnp.float32),
            pltpu.VMEM((NP,), jnp.float32),
        ],
        compiler_params=pltpu.CompilerParams(needs_layout_passes=False),
    )(_k3_body)
    return f(xl0, xl1, esrc, edst, logits, tmax, mself)


def _k4_body(a0_ref, a1_ref, sp_ref, xl0_ref, xl1_ref, xr0_ref, xr1_ref,
             tmax_ref, m_ref, sum_ref, we_ref, att_ref, bias_ref, out_ref):
    M = jnp.maximum(jnp.max(tmax_ref[...]), jnp.max(m_ref[...]))
    crow = jnp.dot(sum_ref[...] * (1.0 / E), we_ref[...],
                   preferred_element_type=jnp.float32)
    xl = jnp.concatenate([xl0_ref[...], xl1_ref[...]], axis=1)
    xr = jnp.concatenate([xr0_ref[...], xr1_ref[...]], axis=1)
    z = xl + xr + crow
    z = jnp.maximum(z, NEG * z)
    ls = jnp.sum(z * att_ref[...], axis=1, keepdims=True)
    exs = jnp.exp(ls - M)
    ssum = jnp.sum(sp_ref[...], axis=0)[:, None] + exs
    num = jnp.concatenate([a0_ref[...], a1_ref[...]], axis=1) + exs * xl
    out_ref[...] = jnp.maximum(num / ssum + bias_ref[...], 0.0)


def _k4(accout, sparts2, xl0, xl1, xr0, xr1, tmax, mself, sumea, W_e,
        att2, bias2):
    return pl.pallas_call(
        _k4_body,
        grid=(NP // NBLK,),
        in_specs=[
            pl.BlockSpec((NBLK, DH), lambda i: (i, 0)),
            pl.BlockSpec((NBLK, DH), lambda i: (i + NP // NBLK, 0)),
            pl.BlockSpec((16, NBLK), lambda i: (0, i)),
            pl.BlockSpec((NBLK, DH), lambda i: (i, 0)),
            pl.BlockSpec((NBLK, DH), lambda i: (i, 0)),
            pl.BlockSpec((NBLK, DH), lambda i: (i, 0)),
            pl.BlockSpec((NBLK, DH), lambda i: (i, 0)),
            pl.BlockSpec((NT2, 16), lambda i: (0, 0)),
            pl.BlockSpec((8, 128), lambda i: (0, 0)),
            pl.BlockSpec((1, DE), lambda i: (0, 0)),
            pl.BlockSpec((DE, D), lambda i: (0, 0)),
            pl.BlockSpec((1, D), lambda i: (0, 0)),
            pl.BlockSpec((1, D), lambda i: (0, 0)),
        ],
        out_specs=pl.BlockSpec((NBLK, D), lambda i: (i, 0)),
        out_shape=jax.ShapeDtypeStruct((NP, D), jnp.float32),
    )(accout, accout, sparts2, xl0, xl1, xr0, xr1, tmax, mself, sumea, W_e,
      att2, bias2)


def kernel(x, edge_index, edge_attr, return_attention_weights,
           W_l, W_r, W_e, att, bias):
    ei = edge_index.astype(jnp.int32)
    esrc = ei[0]
    edst = ei[1]
    att2 = att.reshape(1, D)
    bias2 = bias.reshape(1, D)
    xp = jnp.zeros((NP, D), jnp.float32).at[:N].set(x)
    ee, sumea = _kee(edge_attr, W_e)
    xl0, xl1, xr0, xr1, mself = _k1(xp, W_l, W_r, W_e, sumea, att2)
    logits, tmax = _k2(xl0, xl1, xr0, xr1, ee, esrc, edst, att)
    accout, sparts = _k3(xl0, xl1, esrc, edst, logits, tmax, mself)
    sparts2 = sparts.reshape(16, NP)
    out = _k4(accout, sparts2, xl0, xl1, xr0, xr1, tmax, mself, sumea, W_e,
              att2, bias2)
    return out[:N]


# speedup vs baseline: 3.6890x; 1.7376x over previous
import functools

import jax
import jax.numpy as jnp
from jax import lax
from jax.experimental import pallas as pl
from jax.experimental.pallas import tpu as pltpu
from jax.experimental.pallas import tpu_sc as plsc

N = 10000
E = 160000
D = 256
DH = 128
DE = 16
NEG = 0.2
NEG_INF = -3e38
NP = 12800
NBLK = 512

NT2 = 32
EPT2 = E // NT2
K2B = 40
NB2 = EPT2 // K2B

EPT3 = E // 16
K3B = 16
NB3 = EPT3 // K3B


def _kee_body(ea_ref, we_ref, ee_ref, sum_ref):
    i = pl.program_id(0)
    ea = ea_ref[...]
    ee_ref[...] = jnp.dot(ea, we_ref[...], preferred_element_type=jnp.float32)

    @pl.when(i == 0)
    def _():
        sum_ref[...] = jnp.zeros((1, DE), jnp.float32)

    sum_ref[...] += jnp.sum(ea, axis=0, keepdims=True)


def _kee(edge_attr, W_e):
    blk = 2000
    return pl.pallas_call(
        _kee_body,
        grid=(E // blk,),
        in_specs=[
            pl.BlockSpec((blk, DE), lambda i: (i, 0)),
            pl.BlockSpec((DE, D), lambda i: (0, 0)),
        ],
        out_specs=[
            pl.BlockSpec((blk, D), lambda i: (i, 0)),
            pl.BlockSpec((1, DE), lambda i: (0, 0)),
        ],
        out_shape=[
            jax.ShapeDtypeStruct((E, D), jnp.float32),
            jax.ShapeDtypeStruct((1, DE), jnp.float32),
        ],
    )(edge_attr, W_e)


def _k1_body(x_ref, wl_ref, wr_ref, we_ref, sum_ref, att_ref,
             xl0_ref, xl1_ref, xr0_ref, xr1_ref, m_ref):
    i = pl.program_id(0)
    xv = x_ref[...]
    xl = jnp.dot(xv, wl_ref[...], preferred_element_type=jnp.float32)
    xr = jnp.dot(xv, wr_ref[...], preferred_element_type=jnp.float32)
    xl0_ref[...] = xl[:, :DH]
    xl1_ref[...] = xl[:, DH:]
    xr0_ref[...] = xr[:, :DH]
    xr1_ref[...] = xr[:, DH:]
    crow = jnp.dot(sum_ref[...] * (1.0 / E), we_ref[...],
                   preferred_element_type=jnp.float32)
    z = xl + xr + crow
    z = jnp.maximum(z, NEG * z)
    ls = jnp.sum(z * att_ref[...], axis=1)
    m = jnp.max(ls)

    @pl.when(i == 0)
    def _():
        m_ref[...] = jnp.full((8, 128), NEG_INF, jnp.float32)

    m_ref[...] = jnp.maximum(m_ref[...], m)


def _k1(xp, W_l, W_r, W_e, sumea, att2):
    return pl.pallas_call(
        _k1_body,
        grid=(NP // NBLK,),
        in_specs=[
            pl.BlockSpec((NBLK, D), lambda i: (i, 0)),
            pl.BlockSpec((D, D), lambda i: (0, 0)),
            pl.BlockSpec((D, D), lambda i: (0, 0)),
            pl.BlockSpec((DE, D), lambda i: (0, 0)),
            pl.BlockSpec((1, DE), lambda i: (0, 0)),
            pl.BlockSpec((1, D), lambda i: (0, 0)),
        ],
        out_specs=[
            pl.BlockSpec((NBLK, DH), lambda i: (i, 0)),
            pl.BlockSpec((NBLK, DH), lambda i: (i, 0)),
            pl.BlockSpec((NBLK, DH), lambda i: (i, 0)),
            pl.BlockSpec((NBLK, DH), lambda i: (i, 0)),
            pl.BlockSpec((8, 128), lambda i: (0, 0)),
        ],
        out_shape=[
            jax.ShapeDtypeStruct((NP, DH), jnp.float32),
            jax.ShapeDtypeStruct((NP, DH), jnp.float32),
            jax.ShapeDtypeStruct((NP, DH), jnp.float32),
            jax.ShapeDtypeStruct((NP, DH), jnp.float32),
            jax.ShapeDtypeStruct((8, 128), jnp.float32),
        ],
    )(xp, W_l, W_r, W_e, sumea, att2)


def _k2_body(xl0, xl1, xr0, xr1, ee, esrc, edst, att,
             logits, tmax,
             srcbA, srcbB, dstbA, dstbB,
             bxl0A, bxl0B, bxl1A, bxl1B, bxr0A, bxr0B, bxr1A, bxr1B,
             beeA, beeB, lbufA, lbufB, attv, mxb, tbuf,
             s_idx, s_g0, s_g1, s_g2, s_g3, s_ee, s_logA, s_logB):
    c = lax.axis_index("c")
    s = lax.axis_index("s")
    wid = s * 2 + c
    base = wid * EPT2
    srcb = (srcbA, srcbB)
    dstb = (dstbA, dstbB)
    bxl0b = (bxl0A, bxl0B)
    bxl1b = (bxl1A, bxl1B)
    bxr0b = (bxr0A, bxr0B)
    bxr1b = (bxr1A, bxr1B)
    beeb = (beeA, beeB)
    lbufb = (lbufA, lbufB)

    pltpu.sync_copy(att, attv)
    attc = [attv[pl.ds(16 * k, 16)] for k in range(16)]
    iota = lax.broadcasted_iota(jnp.int32, (16,), 0)

    def idx_descs(g, p):
        return (pltpu.make_async_copy(esrc.at[pl.ds(base + g * K2B, K2B)],
                                      srcb[p], s_idx),
                pltpu.make_async_copy(edst.at[pl.ds(base + g * K2B, K2B)],
                                      dstb[p], s_idx))

    def gather_descs(g, p):
        e0 = base + g * K2B
        return (pltpu.make_async_copy(xl0.at[srcb[p]], bxl0b[p], s_g0),
                pltpu.make_async_copy(xl1.at[srcb[p]], bxl1b[p], s_g1),
                pltpu.make_async_copy(xr0.at[dstb[p]], bxr0b[p], s_g2),
                pltpu.make_async_copy(xr1.at[dstb[p]], bxr1b[p], s_g3),
                pltpu.make_async_copy(ee.at[pl.ds(e0, K2B)], beeb[p], s_ee))

    def log_desc(g, p):
        e0 = base + g * K2B
        return pltpu.make_async_copy(lbufb[p].at[pl.ds(0, K2B)],
                                     logits.at[pl.ds(e0, K2B)],
                                     s_logA if p == 0 else s_logB)

    def compute(g, p, mx16):
        bl0, bl1, br0, br1, be, lb = (bxl0b[p], bxl1b[p], bxr0b[p],
                                      bxr1b[p], beeb[p], lbufb[p])
        for gg in range(3):
            ne = 16 if gg < 2 else K2B - 32

            def edge(je, _):
                j = gg * 16 + je
                t = jnp.zeros((16,), jnp.float32)
                for k in range(8):
                    z = (bl0[j, pl.ds(16 * k, 16)]
                         + br0[j, pl.ds(16 * k, 16)]
                         + be[j, pl.ds(16 * k, 16)])
                    z = jnp.maximum(z, NEG * z)
                    t = t + attc[k] * z
                for k in range(8):
                    z = (bl1[j, pl.ds(16 * k, 16)]
                         + br1[j, pl.ds(16 * k, 16)]
                         + be[j, pl.ds(DH + 16 * k, 16)])
                    z = jnp.maximum(z, NEG * z)
                    t = t + attc[8 + k] * z
                plsc.store_scatter(tbuf, [iota * 16 + je], t)
                return 0

            lax.fori_loop(0, ne, edge, 0)
            colsum = tbuf[pl.ds(0, 16)]
            for r in range(1, 16):
                colsum = colsum + tbuf[pl.ds(16 * r, 16)]
            lb[pl.ds(16 * gg, 16)] = colsum
            mx16 = jnp.maximum(mx16, colsum)
        return mx16

    def body_half(g, p, mx16):
        for d in gather_descs(g, p):
            d.wait()
        for d in idx_descs(g + 1, 1 - p):
            d.wait()
        for d in gather_descs(g + 1, 1 - p):
            d.start()
        gnxt = jnp.minimum(g + 2, NB2 - 1)
        for d in idx_descs(gnxt, p):
            d.start()

        @pl.when(g >= 2)
        def _():
            log_desc(g - 2, p).wait()

        mx16 = compute(g, p, mx16)
        log_desc(g, p).start()
        return mx16

    for d in idx_descs(0, 0):
        d.start()
    for d in idx_descs(0, 0):
        d.wait()
    for d in gather_descs(0, 0):
        d.start()
    for d in idx_descs(1, 1):
        d.start()

    def pair(gp, mx16):
        g = gp * 2
        mx16 = body_half(g, 0, mx16)
        mx16 = body_half(g + 1, 1, mx16)
        return mx16

    mx16 = lax.fori_loop(0, (NB2 - 1) // 2, pair,
                         jnp.full((16,), NEG_INF, jnp.float32))

    gl = NB2 - 1
    for d in gather_descs(gl, 0):
        d.wait()
    log_desc(gl - 2, 0).wait()
    mx16 = compute(gl, 0, mx16)
    log_desc(gl, 0).start()
    for d in idx_descs(gl, 1):
        d.wait()
    log_desc(gl - 1, 1).wait()
    log_desc(gl, 0).wait()

    mxb[...] = mx16
    pltpu.sync_copy(mxb, tmax.at[wid])


def _k2(xl0, xl1, xr0, xr1, ee, esrc, edst, att):
    mesh = plsc.VectorSubcoreMesh(core_axis_name="c", subcore_axis_name="s")
    f = functools.partial(
        pl.kernel,
        out_type=[
            jax.ShapeDtypeStruct((E,), jnp.float32),
            jax.ShapeDtypeStruct((NT2, 16), jnp.float32),
        ],
        mesh=mesh,
        scratch_types=(
            [pltpu.VMEM((K2B,), jnp.int32)] * 4
            + [pltpu.VMEM((K2B, DH), jnp.float32)] * 8
            + [pltpu.VMEM((K2B, D), jnp.float32)] * 2
            + [pltpu.VMEM((48,), jnp.float32)] * 2
            + [pltpu.VMEM((D,), jnp.float32),
               pltpu.VMEM((16,), jnp.float32),
               pltpu.VMEM((D,), jnp.float32)]
            + [pltpu.SemaphoreType.DMA] * 8
        ),
        compiler_params=pltpu.CompilerParams(needs_layout_passes=False),
    )(_k2_body)
    return f(xl0, xl1, xr0, xr1, ee, esrc, edst, att)


def _k3_body(xl0, xl1, esrc, edst, logits, tmax, mself,
             accout, sparts,
             acc, srcbA, srcbB, dstbA, dstbB, dstsA, dstsB,
             gbA, gbB, stageA, stageB,
             lbA, lbB, tb, msb, stab,
             s_idx, s_gb, s_lg, s_scA, s_scB):
    c = lax.axis_index("c")
    s = lax.axis_index("s")
    iota = lax.broadcasted_iota(jnp.int32, (16,), 0)
    srcb = (srcbA, srcbB)
    dstb = (dstbA, dstbB)
    dsts = (dstsA, dstsB)
    gbb = (gbA, gbB)
    stage = (stageA, stageB)
    lbb = (lbA, lbB)
    s_sc = (s_scA, s_scB)

    pltpu.sync_copy(tmax, tb)
    pltpu.sync_copy(mself.at[0, pl.ds(0, 16)], msb)
    m16 = msb[...]
    for r in range(NT2):
        m16 = jnp.maximum(m16, tb[r])
    for sh in (8, 4, 2, 1):
        msb[...] = m16
        m16 = jnp.maximum(m16, plsc.load_gather(msb, [(iota + sh) & 15]))

    z16 = jnp.zeros((16,), jnp.float32)

    def zrow(i, _):
        for k in range(DH // 16):
            stageA[i, pl.ds(16 * k, 16)] = z16
        return 0

    lax.fori_loop(0, K3B, zrow, 0)

    def zacc(q, _):
        pltpu.sync_copy(stageA, acc.at[pl.ds(s * 800 + q * K3B, K3B)])
        return 0

    lax.fori_loop(0, 800 // K3B, zacc, 0)

    @pl.when(c == 0)
    def _():
        def zs(i, _):
            stab[pl.ds(16 * i, 16)] = z16
            return 0

        lax.fori_loop(0, NP // 16, zs, 0)

    plsc.subcore_barrier()

    ebase = s * EPT3

    def run(xtab, do_s):
        def idx_descs(g, p):
            e0 = ebase + g * K3B
            return (pltpu.make_async_copy(esrc.at[pl.ds(e0, K3B)],
                                          srcb[p], s_idx),
                    pltpu.make_async_copy(edst.at[pl.ds(e0, K3B)],
                                          dstb[p], s_idx))

        def gather_descs(g, p):
            e0 = ebase + g * K3B
            return (pltpu.make_async_copy(xtab.at[srcb[p]], gbb[p], s_gb),
                    pltpu.make_async_copy(logits.at[pl.ds(e0, K3B)],
                                          lbb[p], s_lg))

        def scat_desc(p):
            return pltpu.make_async_copy(stage[p], acc.at[dsts[p]], s_sc[p])

        def scat_start(p):
            pltpu.async_copy(stage[p], acc.at[dsts[p]], s_sc[p], add=True)

        def compute(g, p):
            lb, gb, stg = lbb[p], gbb[p], stage[p]
            dsts[p][...] = dstb[p][...]
            exb = lb
            exb[...] = jnp.exp(lb[...] - m16)

            if do_s:
                dst16 = dstb[p][...]
                ex16 = exb[...]
                for r in range(16):
                    plsc.addupdate_scatter(stab, [dst16], ex16,
                                           mask=iota == r)

            def edge(j, _):
                exs = plsc.load_gather(exb, [jnp.full((16,), j, jnp.int32)])
                for k in range(DH // 16):
                    stg[j, pl.ds(16 * k, 16)] = gb[j, pl.ds(16 * k, 16)] * exs
                return 0

            lax.fori_loop(0, K3B, edge, 0)

        def body_half(g, p):
            for d in gather_descs(g, p):
                d.wait()
            for d in idx_descs(g + 1, 1 - p):
                d.wait()
            for d in gather_descs(g + 1, 1 - p):
                d.start()

            @pl.when(g >= 2)
            def _():
                scat_desc(p).wait()

            compute(g, p)
            scat_start(p)
            gnxt = jnp.minimum(g + 2, NB3 - 1)
            for d in idx_descs(gnxt, p):
                d.start()

        for d in idx_descs(0, 0):
            d.start()
        for d in idx_descs(0, 0):
            d.wait()
        for d in gather_descs(0, 0):
            d.start()
        for d in idx_descs(1, 1):
            d.start()

        def pair(gp, _):
            g = gp * 2
            body_half(g, 0)
            body_half(g + 1, 1)
            return 0

        lax.fori_loop(0, (NB3 - 1) // 2, pair, 0)

        gl = NB3 - 1
        for d in gather_descs(gl, 0):
            d.wait()
        scat_desc(0).wait()
        compute(jnp.int32(gl), 0)
        scat_start(0)
        for d in idx_descs(gl, 1):
            d.wait()
        scat_desc(1).wait()
        scat_desc(0).wait()

    @pl.when(c == 0)
    def _():
        run(xl0, True)

    @pl.when(c == 1)
    def _():
        run(xl1, False)

    plsc.subcore_barrier()
    pltpu.sync_copy(acc.at[pl.ds(s * 800, 800)],
                    accout.at[pl.ds(c * NP + s * 800, 800)])

    @pl.when(c == 0)
    def _():
        pltpu.sync_copy(stab, sparts.at[pl.ds(s * NP, NP)])


def _k3(xl0, xl1, esrc, edst, logits, tmax, mself):
    mesh = plsc.VectorSubcoreMesh(core_axis_name="c", subcore_axis_name="s")
    f = functools.partial(
        pl.kernel,
        out_type=[
            jax.ShapeDtypeStruct((2 * NP, DH), jnp.float32),
            jax.ShapeDtypeStruct((16 * NP,), jnp.float32),
        ],
        mesh=mesh,
        scratch_types=(
            [pltpu.VMEM_SHARED((NP, DH), jnp.float32)]
            + [pltpu.VMEM((K3B,), jnp.int32)] * 6
            + [pltpu.VMEM((K3B, DH), jnp.float32)] * 4
            + [pltpu.VMEM((K3B,), jnp.float32)] * 2
            + [pltpu.VMEM((NT2, 16), jnp.float32),
               pltpu.VMEM((16,), jnp.float32),
               pltpu.VMEM((NP,), jnp.float32)]
            + [pltpu.SemaphoreType.DMA] * 5
        ),
        compiler_params=pltpu.CompilerParams(needs_layout_passes=False),
    )(_k3_body)
    return f(xl0, xl1, esrc, edst, logits, tmax, mself)


def _k4_body(a0_ref, a1_ref, sp_ref, xl0_ref, xl1_ref, xr0_ref, xr1_ref,
             tmax_ref, m_ref, sum_ref, we_ref, att_ref, bias_ref, out_ref):
    M = jnp.maximum(jnp.max(tmax_ref[...]), jnp.max(m_ref[...]))
    crow = jnp.dot(sum_ref[...] * (1.0 / E), we_ref[...],
                   preferred_element_type=jnp.float32)
    xl = jnp.concatenate([xl0_ref[...], xl1_ref[...]], axis=1)
    xr = jnp.concatenate([xr0_ref[...], xr1_ref[...]], axis=1)
    z = xl + xr + crow
    z = jnp.maximum(z, NEG * z)
    ls = jnp.sum(z * att_ref[...], axis=1, keepdims=True)
    exs = jnp.exp(ls - M)
    ssum = jnp.sum(sp_ref[...], axis=0)[:, None] + exs
    num = jnp.concatenate([a0_ref[...], a1_ref[...]], axis=1) + exs * xl
    out_ref[...] = jnp.maximum(num / ssum + bias_ref[...], 0.0)


def _k4(accout, sparts2, xl0, xl1, xr0, xr1, tmax, mself, sumea, W_e,
        att2, bias2):
    return pl.pallas_call(
        _k4_body,
        grid=(NP // NBLK,),
        in_specs=[
            pl.BlockSpec((NBLK, DH), lambda i: (i, 0)),
            pl.BlockSpec((NBLK, DH), lambda i: (i + NP // NBLK, 0)),
            pl.BlockSpec((16, NBLK), lambda i: (0, i)),
            pl.BlockSpec((NBLK, DH), lambda i: (i, 0)),
            pl.BlockSpec((NBLK, DH), lambda i: (i, 0)),
            pl.BlockSpec((NBLK, DH), lambda i: (i, 0)),
            pl.BlockSpec((NBLK, DH), lambda i: (i, 0)),
            pl.BlockSpec((NT2, 16), lambda i: (0, 0)),
            pl.BlockSpec((8, 128), lambda i: (0, 0)),
            pl.BlockSpec((1, DE), lambda i: (0, 0)),
            pl.BlockSpec((DE, D), lambda i: (0, 0)),
            pl.BlockSpec((1, D), lambda i: (0, 0)),
            pl.BlockSpec((1, D), lambda i: (0, 0)),
        ],
        out_specs=pl.BlockSpec((NBLK, D), lambda i: (i, 0)),
        out_shape=jax.ShapeDtypeStruct((NP, D), jnp.float32),
    )(accout, accout, sparts2, xl0, xl1, xr0, xr1, tmax, mself, sumea, W_e,
      att2, bias2)


def kernel(x, edge_index, edge_attr, return_attention_weights,
           W_l, W_r, W_e, att, bias):
    ei = edge_index.astype(jnp.int32)
    esrc = ei[0]
    edst = ei[1]
    att2 = att.reshape(1, D)
    bias2 = bias.reshape(1, D)
    xp = jnp.zeros((NP, D), jnp.float32).at[:N].set(x)
    ee, sumea = _kee(edge_attr, W_e)
    xl0, xl1, xr0, xr1, mself = _k1(xp, W_l, W_r, W_e, sumea, att2)
    logits, tmax = _k2(xl0, xl1, xr0, xr1, ee, esrc, edst, att)
    accout, sparts = _k3(xl0, xl1, esrc, edst, logits, tmax, mself)
    sparts2 = sparts.reshape(16, NP)
    out = _k4(accout, sparts2, xl0, xl1, xr0, xr1, tmax, mself, sumea, W_e,
              att2, bias2)
    return out[:N]


# speedup vs baseline: 4.2899x; 1.1629x over previous
import functools

import jax
import jax.numpy as jnp
from jax import lax
from jax.experimental import pallas as pl
from jax.experimental.pallas import tpu as pltpu
from jax.experimental.pallas import tpu_sc as plsc

N = 10000
E = 160000
D = 256
DH = 128
DE = 16
NEG = 0.2
NEG_INF = -3e38
NP = 10240
NBLK = 512

NT2 = 32
EPT2 = E // NT2
K2B = 40
NB2 = EPT2 // K2B

EPT3 = E // 16
K3B = 40
NB3 = EPT3 // K3B


def _kee_body(ea_ref, we_ref, ee_ref, sum_ref):
    i = pl.program_id(0)
    ea = ea_ref[...]
    ee_ref[...] = jnp.dot(ea, we_ref[...], preferred_element_type=jnp.float32)

    @pl.when(i == 0)
    def _():
        sum_ref[...] = jnp.zeros((1, DE), jnp.float32)

    sum_ref[...] += jnp.sum(ea, axis=0, keepdims=True)


def _kee(edge_attr, W_e):
    blk = 2000
    return pl.pallas_call(
        _kee_body,
        grid=(E // blk,),
        in_specs=[
            pl.BlockSpec((blk, DE), lambda i: (i, 0)),
            pl.BlockSpec((DE, D), lambda i: (0, 0)),
        ],
        out_specs=[
            pl.BlockSpec((blk, D), lambda i: (i, 0)),
            pl.BlockSpec((1, DE), lambda i: (0, 0)),
        ],
        out_shape=[
            jax.ShapeDtypeStruct((E, D), jnp.float32),
            jax.ShapeDtypeStruct((1, DE), jnp.float32),
        ],
    )(edge_attr, W_e)


def _k1_body(x_ref, wl_ref, wr_ref, we_ref, sum_ref, att_ref,
             xl0_ref, xl1_ref, xr0_ref, xr1_ref, m_ref):
    i = pl.program_id(0)
    xv = x_ref[...]
    xl = jnp.dot(xv, wl_ref[...], preferred_element_type=jnp.float32)
    xr = jnp.dot(xv, wr_ref[...], preferred_element_type=jnp.float32)
    xl0_ref[...] = xl[:, :DH]
    xl1_ref[...] = xl[:, DH:]
    xr0_ref[...] = xr[:, :DH]
    xr1_ref[...] = xr[:, DH:]
    crow = jnp.dot(sum_ref[...] * (1.0 / E), we_ref[...],
                   preferred_element_type=jnp.float32)
    z = xl + xr + crow
    z = jnp.maximum(z, NEG * z)
    ls = jnp.sum(z * att_ref[...], axis=1)
    m = jnp.max(ls)

    @pl.when(i == 0)
    def _():
        m_ref[...] = jnp.full((8, 128), NEG_INF, jnp.float32)

    m_ref[...] = jnp.maximum(m_ref[...], m)


def _k1(xp, W_l, W_r, W_e, sumea, att2):
    return pl.pallas_call(
        _k1_body,
        grid=(NP // NBLK,),
        in_specs=[
            pl.BlockSpec((NBLK, D), lambda i: (i, 0)),
            pl.BlockSpec((D, D), lambda i: (0, 0)),
            pl.BlockSpec((D, D), lambda i: (0, 0)),
            pl.BlockSpec((DE, D), lambda i: (0, 0)),
            pl.BlockSpec((1, DE), lambda i: (0, 0)),
            pl.BlockSpec((1, D), lambda i: (0, 0)),
        ],
        out_specs=[
            pl.BlockSpec((NBLK, DH), lambda i: (i, 0)),
            pl.BlockSpec((NBLK, DH), lambda i: (i, 0)),
            pl.BlockSpec((NBLK, DH), lambda i: (i, 0)),
            pl.BlockSpec((NBLK, DH), lambda i: (i, 0)),
            pl.BlockSpec((8, 128), lambda i: (0, 0)),
        ],
        out_shape=[
            jax.ShapeDtypeStruct((NP, DH), jnp.float32),
            jax.ShapeDtypeStruct((NP, DH), jnp.float32),
            jax.ShapeDtypeStruct((NP, DH), jnp.float32),
            jax.ShapeDtypeStruct((NP, DH), jnp.float32),
            jax.ShapeDtypeStruct((8, 128), jnp.float32),
        ],
    )(xp, W_l, W_r, W_e, sumea, att2)


def _k2_body(xl0, xl1, xr0, xr1, ee, esrc, edst, att,
             logits, tmax,
             srcbA, srcbB, dstbA, dstbB,
             bxl0A, bxl0B, bxl1A, bxl1B, bxr0A, bxr0B, bxr1A, bxr1B,
             beeA, beeB, lbufA, lbufB, attv, mxb, tbuf,
             s_idx, s_g0, s_g1, s_g2, s_g3, s_ee, s_logA, s_logB):
    c = lax.axis_index("c")
    s = lax.axis_index("s")
    wid = s * 2 + c
    base = wid * EPT2
    srcb = (srcbA, srcbB)
    dstb = (dstbA, dstbB)
    bxl0b = (bxl0A, bxl0B)
    bxl1b = (bxl1A, bxl1B)
    bxr0b = (bxr0A, bxr0B)
    bxr1b = (bxr1A, bxr1B)
    beeb = (beeA, beeB)
    lbufb = (lbufA, lbufB)

    pltpu.sync_copy(att, attv)
    attc = [attv[pl.ds(16 * k, 16)] for k in range(16)]
    iota = lax.broadcasted_iota(jnp.int32, (16,), 0)

    def idx_descs(g, p):
        return (pltpu.make_async_copy(esrc.at[pl.ds(base + g * K2B, K2B)],
                                      srcb[p], s_idx),
                pltpu.make_async_copy(edst.at[pl.ds(base + g * K2B, K2B)],
                                      dstb[p], s_idx))

    def gather_descs(g, p):
        e0 = base + g * K2B
        return (pltpu.make_async_copy(xl0.at[srcb[p]], bxl0b[p], s_g0),
                pltpu.make_async_copy(xl1.at[srcb[p]], bxl1b[p], s_g1),
                pltpu.make_async_copy(xr0.at[dstb[p]], bxr0b[p], s_g2),
                pltpu.make_async_copy(xr1.at[dstb[p]], bxr1b[p], s_g3),
                pltpu.make_async_copy(ee.at[pl.ds(e0, K2B)], beeb[p], s_ee))

    def log_desc(g, p):
        e0 = base + g * K2B
        return pltpu.make_async_copy(lbufb[p].at[pl.ds(0, K2B)],
                                     logits.at[pl.ds(e0, K2B)],
                                     s_logA if p == 0 else s_logB)

    def compute(g, p, mx16):
        bl0, bl1, br0, br1, be, lb = (bxl0b[p], bxl1b[p], bxr0b[p],
                                      bxr1b[p], beeb[p], lbufb[p])
        for gg in range(3):
            ne = 16 if gg < 2 else K2B - 32

            def edge(je, _):
                j = gg * 16 + je
                t = jnp.zeros((16,), jnp.float32)
                for k in range(8):
                    z = (bl0[j, pl.ds(16 * k, 16)]
                         + br0[j, pl.ds(16 * k, 16)]
                         + be[j, pl.ds(16 * k, 16)])
                    z = jnp.maximum(z, NEG * z)
                    t = t + attc[k] * z
                for k in range(8):
                    z = (bl1[j, pl.ds(16 * k, 16)]
                         + br1[j, pl.ds(16 * k, 16)]
                         + be[j, pl.ds(DH + 16 * k, 16)])
                    z = jnp.maximum(z, NEG * z)
                    t = t + attc[8 + k] * z
                plsc.store_scatter(tbuf, [iota * 16 + je], t)
                return 0

            lax.fori_loop(0, ne, edge, 0)
            colsum = tbuf[pl.ds(0, 16)]
            for r in range(1, 16):
                colsum = colsum + tbuf[pl.ds(16 * r, 16)]
            lb[pl.ds(16 * gg, 16)] = colsum
            mx16 = jnp.maximum(mx16, colsum)
        return mx16

    def body_half(g, p, mx16):
        for d in gather_descs(g, p):
            d.wait()
        for d in idx_descs(g + 1, 1 - p):
            d.wait()
        for d in gather_descs(g + 1, 1 - p):
            d.start()
        gnxt = jnp.minimum(g + 2, NB2 - 1)
        for d in idx_descs(gnxt, p):
            d.start()

        @pl.when(g >= 2)
        def _():
            log_desc(g - 2, p).wait()

        mx16 = compute(g, p, mx16)
        log_desc(g, p).start()
        return mx16

    for d in idx_descs(0, 0):
        d.start()
    for d in idx_descs(0, 0):
        d.wait()
    for d in gather_descs(0, 0):
        d.start()
    for d in idx_descs(1, 1):
        d.start()

    def pair(gp, mx16):
        g = gp * 2
        mx16 = body_half(g, 0, mx16)
        mx16 = body_half(g + 1, 1, mx16)
        return mx16

    mx16 = lax.fori_loop(0, (NB2 - 1) // 2, pair,
                         jnp.full((16,), NEG_INF, jnp.float32))

    gl = NB2 - 1
    for d in gather_descs(gl, 0):
        d.wait()
    log_desc(gl - 2, 0).wait()
    mx16 = compute(gl, 0, mx16)
    log_desc(gl, 0).start()
    for d in idx_descs(gl, 1):
        d.wait()
    log_desc(gl - 1, 1).wait()
    log_desc(gl, 0).wait()

    mxb[...] = mx16
    pltpu.sync_copy(mxb, tmax.at[wid])


def _k2(xl0, xl1, xr0, xr1, ee, esrc, edst, att):
    mesh = plsc.VectorSubcoreMesh(core_axis_name="c", subcore_axis_name="s")
    f = functools.partial(
        pl.kernel,
        out_type=[
            jax.ShapeDtypeStruct((E,), jnp.float32),
            jax.ShapeDtypeStruct((NT2, 16), jnp.float32),
        ],
        mesh=mesh,
        scratch_types=(
            [pltpu.VMEM((K2B,), jnp.int32)] * 4
            + [pltpu.VMEM((K2B, DH), jnp.float32)] * 8
            + [pltpu.VMEM((K2B, D), jnp.float32)] * 2
            + [pltpu.VMEM((48,), jnp.float32)] * 2
            + [pltpu.VMEM((D,), jnp.float32),
               pltpu.VMEM((16,), jnp.float32),
               pltpu.VMEM((D,), jnp.float32)]
            + [pltpu.SemaphoreType.DMA] * 8
        ),
        compiler_params=pltpu.CompilerParams(needs_layout_passes=False),
    )(_k2_body)
    return f(xl0, xl1, xr0, xr1, ee, esrc, edst, att)


def _k3_body(xl0, xl1, esrc, edst, logits, tmax, mself,
             accout, sparts,
             acc, srcbA, srcbB, dstbA, dstbB, dstsA, dstsB,
             gbA, gbB, stageA, stageB,
             lbA, lbB, tb, msb, stab,
             s_idx, s_gb, s_lg, s_scA, s_scB):
    c = lax.axis_index("c")
    s = lax.axis_index("s")
    iota = lax.broadcasted_iota(jnp.int32, (16,), 0)
    srcb = (srcbA, srcbB)
    dstb = (dstbA, dstbB)
    dsts = (dstsA, dstsB)
    gbb = (gbA, gbB)
    stage = (stageA, stageB)
    lbb = (lbA, lbB)
    s_sc = (s_scA, s_scB)

    pltpu.sync_copy(tmax, tb)
    pltpu.sync_copy(mself.at[0, pl.ds(0, 16)], msb)
    m16 = msb[...]
    for r in range(NT2):
        m16 = jnp.maximum(m16, tb[r])
    for sh in (8, 4, 2, 1):
        msb[...] = m16
        m16 = jnp.maximum(m16, plsc.load_gather(msb, [(iota + sh) & 15]))

    z16 = jnp.zeros((16,), jnp.float32)

    def zrow(i, _):
        for k in range(DH // 16):
            stageA[i, pl.ds(16 * k, 16)] = z16
        return 0

    lax.fori_loop(0, K3B, zrow, 0)

    def zacc(q, _):
        pltpu.sync_copy(stageA, acc.at[pl.ds(s * 640 + q * K3B, K3B)])
        return 0

    lax.fori_loop(0, 640 // K3B, zacc, 0)

    @pl.when(c == 0)
    def _():
        def zs(i, _):
            stab[pl.ds(16 * i, 16)] = z16
            return 0

        lax.fori_loop(0, NP // 16, zs, 0)

    plsc.subcore_barrier()

    ebase = s * EPT3

    def run(xtab, do_s):
        def idx_descs(g, p):
            e0 = ebase + g * K3B
            return (pltpu.make_async_copy(esrc.at[pl.ds(e0, K3B)],
                                          srcb[p], s_idx),
                    pltpu.make_async_copy(edst.at[pl.ds(e0, K3B)],
                                          dstb[p], s_idx))

        def gather_descs(g, p):
            e0 = ebase + g * K3B
            return (pltpu.make_async_copy(xtab.at[srcb[p]], gbb[p], s_gb),
                    pltpu.make_async_copy(logits.at[pl.ds(e0, K3B)],
                                          lbb[p], s_lg))

        def scat_desc(p):
            return pltpu.make_async_copy(stage[p], acc.at[dsts[p]], s_sc[p])

        def scat_start(p):
            pltpu.async_copy(stage[p], acc.at[dsts[p]], s_sc[p], add=True)

        def compute(g, p):
            lb, gb, stg = lbb[p], gbb[p], stage[p]
            for off in (0, 16, 24):
                dsts[p][pl.ds(off, 16)] = dstb[p][pl.ds(off, 16)]
            exb = lb
            for off in (0, 16, 24):
                exb[pl.ds(off, 16)] = jnp.exp(lb[pl.ds(off, 16)] - m16)

            if do_s:
                for off, lo in ((0, 0), (16, 0), (24, 8)):
                    dst16 = dstb[p][pl.ds(off, 16)]
                    ex16 = exb[pl.ds(off, 16)]
                    for r in range(lo, 16):
                        plsc.addupdate_scatter(stab, [dst16], ex16,
                                               mask=iota == r)

            def edge(j, _):
                exs = plsc.load_gather(exb, [jnp.full((16,), j, jnp.int32)])
                for k in range(DH // 16):
                    stg[j, pl.ds(16 * k, 16)] = gb[j, pl.ds(16 * k, 16)] * exs
                return 0

            lax.fori_loop(0, K3B, edge, 0)

        def body_half(g, p):
            for d in gather_descs(g, p):
                d.wait()
            for d in idx_descs(g + 1, 1 - p):
                d.wait()
            for d in gather_descs(g + 1, 1 - p):
                d.start()

            @pl.when(g >= 2)
            def _():
                scat_desc(p).wait()

            compute(g, p)
            scat_start(p)
            gnxt = jnp.minimum(g + 2, NB3 - 1)
            for d in idx_descs(gnxt, p):
                d.start()

        for d in idx_descs(0, 0):
            d.start()
        for d in idx_descs(0, 0):
            d.wait()
        for d in gather_descs(0, 0):
            d.start()
        for d in idx_descs(1, 1):
            d.start()

        def pair(gp, _):
            g = gp * 2
            body_half(g, 0)
            body_half(g + 1, 1)
            return 0

        lax.fori_loop(0, (NB3 - 2) // 2, pair, 0)

        body_half(jnp.int32(NB3 - 2), 0)
        gl = NB3 - 1
        for d in gather_descs(gl, 1):
            d.wait()
        scat_desc(1).wait()
        compute(jnp.int32(gl), 1)
        scat_start(1)
        for d in idx_descs(gl, 0):
            d.wait()
        scat_desc(0).wait()
        scat_desc(1).wait()

    @pl.when(c == 0)
    def _():
        run(xl0, True)

    @pl.when(c == 1)
    def _():
        run(xl1, False)

    plsc.subcore_barrier()
    pltpu.sync_copy(acc.at[pl.ds(s * 640, 640)],
                    accout.at[pl.ds(c * NP + s * 640, 640)])

    @pl.when(c == 0)
    def _():
        pltpu.sync_copy(stab, sparts.at[pl.ds(s * NP, NP)])


def _k3(xl0, xl1, esrc, edst, logits, tmax, mself):
    mesh = plsc.VectorSubcoreMesh(core_axis_name="c", subcore_axis_name="s")
    f = functools.partial(
        pl.kernel,
        out_type=[
            jax.ShapeDtypeStruct((2 * NP, DH), jnp.float32),
            jax.ShapeDtypeStruct((16 * NP,), jnp.float32),
        ],
        mesh=mesh,
        scratch_types=(
            [pltpu.VMEM_SHARED((NP, DH), jnp.float32)]
            + [pltpu.VMEM((K3B,), jnp.int32)] * 6
            + [pltpu.VMEM((K3B, DH), jnp.float32)] * 4
            + [pltpu.VMEM((K3B,), jnp.float32)] * 2
            + [pltpu.VMEM((NT2, 16), jnp.float32),
               pltpu.VMEM((16,), jnp.float32),
               pltpu.VMEM((NP,), jnp.float32)]
            + [pltpu.SemaphoreType.DMA] * 5
        ),
        compiler_params=pltpu.CompilerParams(needs_layout_passes=False),
    )(_k3_body)
    return f(xl0, xl1, esrc, edst, logits, tmax, mself)


def _k4_body(a0_ref, a1_ref, sp_ref, xl0_ref, xl1_ref, xr0_ref, xr1_ref,
             tmax_ref, m_ref, sum_ref, we_ref, att_ref, bias_ref, out_ref):
    M = jnp.maximum(jnp.max(tmax_ref[...]), jnp.max(m_ref[...]))
    crow = jnp.dot(sum_ref[...] * (1.0 / E), we_ref[...],
                   preferred_element_type=jnp.float32)
    xl = jnp.concatenate([xl0_ref[...], xl1_ref[...]], axis=1)
    xr = jnp.concatenate([xr0_ref[...], xr1_ref[...]], axis=1)
    z = xl + xr + crow
    z = jnp.maximum(z, NEG * z)
    ls = jnp.sum(z * att_ref[...], axis=1, keepdims=True)
    exs = jnp.exp(ls - M)
    ssum = jnp.sum(sp_ref[...], axis=0)[:, None] + exs
    num = jnp.concatenate([a0_ref[...], a1_ref[...]], axis=1) + exs * xl
    out_ref[...] = jnp.maximum(num / ssum + bias_ref[...], 0.0)


def _k4(accout, sparts2, xl0, xl1, xr0, xr1, tmax, mself, sumea, W_e,
        att2, bias2):
    return pl.pallas_call(
        _k4_body,
        grid=(NP // NBLK,),
        in_specs=[
            pl.BlockSpec((NBLK, DH), lambda i: (i, 0)),
            pl.BlockSpec((NBLK, DH), lambda i: (i + NP // NBLK, 0)),
            pl.BlockSpec((16, NBLK), lambda i: (0, i)),
            pl.BlockSpec((NBLK, DH), lambda i: (i, 0)),
            pl.BlockSpec((NBLK, DH), lambda i: (i, 0)),
            pl.BlockSpec((NBLK, DH), lambda i: (i, 0)),
            pl.BlockSpec((NBLK, DH), lambda i: (i, 0)),
            pl.BlockSpec((NT2, 16), lambda i: (0, 0)),
            pl.BlockSpec((8, 128), lambda i: (0, 0)),
            pl.BlockSpec((1, DE), lambda i: (0, 0)),
            pl.BlockSpec((DE, D), lambda i: (0, 0)),
            pl.BlockSpec((1, D), lambda i: (0, 0)),
            pl.BlockSpec((1, D), lambda i: (0, 0)),
        ],
        out_specs=pl.BlockSpec((NBLK, D), lambda i: (i, 0)),
        out_shape=jax.ShapeDtypeStruct((NP, D), jnp.float32),
    )(accout, accout, sparts2, xl0, xl1, xr0, xr1, tmax, mself, sumea, W_e,
      att2, bias2)


def kernel(x, edge_index, edge_attr, return_attention_weights,
           W_l, W_r, W_e, att, bias):
    ei = edge_index.astype(jnp.int32)
    esrc = ei[0]
    edst = ei[1]
    att2 = att.reshape(1, D)
    bias2 = bias.reshape(1, D)
    xp = jnp.zeros((NP, D), jnp.float32).at[:N].set(x)
    ee, sumea = _kee(edge_attr, W_e)
    xl0, xl1, xr0, xr1, mself = _k1(xp, W_l, W_r, W_e, sumea, att2)
    logits, tmax = _k2(xl0, xl1, xr0, xr1, ee, esrc, edst, att)
    accout, sparts = _k3(xl0, xl1, esrc, edst, logits, tmax, mself)
    sparts2 = sparts.reshape(16, NP)
    out = _k4(accout, sparts2, xl0, xl1, xr0, xr1, tmax, mself, sumea, W_e,
              att2, bias2)
    return out[:N]


# speedup vs baseline: 4.3023x; 1.0029x over previous
import functools

import jax
import jax.numpy as jnp
from jax import lax
from jax.experimental import pallas as pl
from jax.experimental.pallas import tpu as pltpu
from jax.experimental.pallas import tpu_sc as plsc

N = 10000
E = 160000
D = 256
DH = 128
DE = 16
NEG = 0.2
NEG_INF = -3e38
NP = 10240
NBLK = 512

NT2 = 32
EPT2 = E // NT2
K2B = 40
NB2 = EPT2 // K2B

EPT3 = E // 16
K3B = 40
NB3 = EPT3 // K3B


def _kee_body(ea_ref, we_ref, ee_ref, sum_ref):
    i = pl.program_id(0)
    ea = ea_ref[...]
    ee_ref[...] = jnp.dot(ea, we_ref[...], preferred_element_type=jnp.float32)

    @pl.when(i == 0)
    def _():
        sum_ref[...] = jnp.zeros((1, DE), jnp.float32)

    sum_ref[...] += jnp.sum(ea, axis=0, keepdims=True)


def _kee(edge_attr, W_e):
    blk = 2000
    return pl.pallas_call(
        _kee_body,
        grid=(E // blk,),
        in_specs=[
            pl.BlockSpec((blk, DE), lambda i: (i, 0)),
            pl.BlockSpec((DE, D), lambda i: (0, 0)),
        ],
        out_specs=[
            pl.BlockSpec((blk, D), lambda i: (i, 0)),
            pl.BlockSpec((1, DE), lambda i: (0, 0)),
        ],
        out_shape=[
            jax.ShapeDtypeStruct((E, D), jnp.float32),
            jax.ShapeDtypeStruct((1, DE), jnp.float32),
        ],
    )(edge_attr, W_e)


def _k1_body(x_ref, wl_ref, wr_ref, we_ref, sum_ref, att_ref,
             xl0_ref, xl1_ref, xr0_ref, xr1_ref, m_ref):
    i = pl.program_id(0)
    xv = x_ref[...]
    xl = jnp.dot(xv, wl_ref[...], preferred_element_type=jnp.float32)
    xr = jnp.dot(xv, wr_ref[...], preferred_element_type=jnp.float32)
    xl0_ref[...] = xl[:, :DH]
    xl1_ref[...] = xl[:, DH:]
    xr0_ref[...] = xr[:, :DH]
    xr1_ref[...] = xr[:, DH:]
    crow = jnp.dot(sum_ref[...] * (1.0 / E), we_ref[...],
                   preferred_element_type=jnp.float32)
    z = xl + xr + crow
    z = jnp.maximum(z, NEG * z)
    ls = jnp.sum(z * att_ref[...], axis=1)
    m = jnp.max(ls)

    @pl.when(i == 0)
    def _():
        m_ref[...] = jnp.full((8, 128), NEG_INF, jnp.float32)

    m_ref[...] = jnp.maximum(m_ref[...], m)


def _k1(xp, W_l, W_r, W_e, sumea, att2):
    return pl.pallas_call(
        _k1_body,
        grid=(NP // NBLK,),
        in_specs=[
            pl.BlockSpec((NBLK, D), lambda i: (i, 0)),
            pl.BlockSpec((D, D), lambda i: (0, 0)),
            pl.BlockSpec((D, D), lambda i: (0, 0)),
            pl.BlockSpec((DE, D), lambda i: (0, 0)),
            pl.BlockSpec((1, DE), lambda i: (0, 0)),
            pl.BlockSpec((1, D), lambda i: (0, 0)),
        ],
        out_specs=[
            pl.BlockSpec((NBLK, DH), lambda i: (i, 0)),
            pl.BlockSpec((NBLK, DH), lambda i: (i, 0)),
            pl.BlockSpec((NBLK, DH), lambda i: (i, 0)),
            pl.BlockSpec((NBLK, DH), lambda i: (i, 0)),
            pl.BlockSpec((8, 128), lambda i: (0, 0)),
        ],
        out_shape=[
            jax.ShapeDtypeStruct((NP, DH), jnp.float32),
            jax.ShapeDtypeStruct((NP, DH), jnp.float32),
            jax.ShapeDtypeStruct((NP, DH), jnp.float32),
            jax.ShapeDtypeStruct((NP, DH), jnp.float32),
            jax.ShapeDtypeStruct((8, 128), jnp.float32),
        ],
    )(xp, W_l, W_r, W_e, sumea, att2)


def _k2_body(xl0, xl1, xr0, xr1, ee, esrc, edst, att,
             logits, tmax,
             srcbA, srcbB, dstbA, dstbB,
             bxl0A, bxl0B, bxl1A, bxl1B, bxr0A, bxr0B, bxr1A, bxr1B,
             beeA, beeB, lbufA, lbufB, attv, mxb, tbuf,
             s_idx, s_g0, s_g1, s_g2, s_g3, s_ee, s_logA, s_logB):
    c = lax.axis_index("c")
    s = lax.axis_index("s")
    wid = s * 2 + c
    base = wid * EPT2
    srcb = (srcbA, srcbB)
    dstb = (dstbA, dstbB)
    bxl0b = (bxl0A, bxl0B)
    bxl1b = (bxl1A, bxl1B)
    bxr0b = (bxr0A, bxr0B)
    bxr1b = (bxr1A, bxr1B)
    beeb = (beeA, beeB)
    lbufb = (lbufA, lbufB)

    pltpu.sync_copy(att, attv)
    attc = [attv[pl.ds(16 * k, 16)] for k in range(16)]
    iota = lax.broadcasted_iota(jnp.int32, (16,), 0)

    def idx_descs(g, p):
        return (pltpu.make_async_copy(esrc.at[pl.ds(base + g * K2B, K2B)],
                                      srcb[p], s_idx),
                pltpu.make_async_copy(edst.at[pl.ds(base + g * K2B, K2B)],
                                      dstb[p], s_idx))

    def gather_descs(g, p):
        e0 = base + g * K2B
        return (pltpu.make_async_copy(xl0.at[srcb[p]], bxl0b[p], s_g0),
                pltpu.make_async_copy(xl1.at[srcb[p]], bxl1b[p], s_g1),
                pltpu.make_async_copy(xr0.at[dstb[p]], bxr0b[p], s_g2),
                pltpu.make_async_copy(xr1.at[dstb[p]], bxr1b[p], s_g3),
                pltpu.make_async_copy(ee.at[pl.ds(e0, K2B)], beeb[p], s_ee))

    def log_desc(g, p):
        e0 = base + g * K2B
        return pltpu.make_async_copy(lbufb[p].at[pl.ds(0, K2B)],
                                     logits.at[pl.ds(e0, K2B)],
                                     s_logA if p == 0 else s_logB)

    def compute(g, p, mx16):
        bl0, bl1, br0, br1, be, lb = (bxl0b[p], bxl1b[p], bxr0b[p],
                                      bxr1b[p], beeb[p], lbufb[p])
        for gg in range(3):
            ne = 16 if gg < 2 else K2B - 32

            def edge(je, _):
                j = gg * 16 + je
                t = jnp.zeros((16,), jnp.float32)
                for k in range(8):
                    z = (bl0[j, pl.ds(16 * k, 16)]
                         + br0[j, pl.ds(16 * k, 16)]
                         + be[j, pl.ds(16 * k, 16)])
                    z = jnp.maximum(z, NEG * z)
                    t = t + attc[k] * z
                for k in range(8):
                    z = (bl1[j, pl.ds(16 * k, 16)]
                         + br1[j, pl.ds(16 * k, 16)]
                         + be[j, pl.ds(DH + 16 * k, 16)])
                    z = jnp.maximum(z, NEG * z)
                    t = t + attc[8 + k] * z
                plsc.store_scatter(tbuf, [iota * 16 + je], t)
                return 0

            lax.fori_loop(0, ne, edge, 0)
            colsum = tbuf[pl.ds(0, 16)]
            for r in range(1, 16):
                colsum = colsum + tbuf[pl.ds(16 * r, 16)]
            lb[pl.ds(16 * gg, 16)] = colsum
            mx16 = jnp.maximum(mx16, colsum)
        return mx16

    def body_half(g, p, mx16):
        for d in gather_descs(g, p):
            d.wait()
        for d in idx_descs(g + 1, 1 - p):
            d.wait()
        for d in gather_descs(g + 1, 1 - p):
            d.start()
        gnxt = jnp.minimum(g + 2, NB2 - 1)
        for d in idx_descs(gnxt, p):
            d.start()

        @pl.when(g >= 2)
        def _():
            log_desc(g - 2, p).wait()

        mx16 = compute(g, p, mx16)
        log_desc(g, p).start()
        return mx16

    for d in idx_descs(0, 0):
        d.start()
    for d in idx_descs(0, 0):
        d.wait()
    for d in gather_descs(0, 0):
        d.start()
    for d in idx_descs(1, 1):
        d.start()

    def pair(gp, mx16):
        g = gp * 2
        mx16 = body_half(g, 0, mx16)
        mx16 = body_half(g + 1, 1, mx16)
        return mx16

    mx16 = lax.fori_loop(0, (NB2 - 1) // 2, pair,
                         jnp.full((16,), NEG_INF, jnp.float32))

    gl = NB2 - 1
    for d in gather_descs(gl, 0):
        d.wait()
    log_desc(gl - 2, 0).wait()
    mx16 = compute(gl, 0, mx16)
    log_desc(gl, 0).start()
    for d in idx_descs(gl, 1):
        d.wait()
    log_desc(gl - 1, 1).wait()
    log_desc(gl, 0).wait()

    mxb[...] = mx16
    pltpu.sync_copy(mxb, tmax.at[wid])


def _k2(xl0, xl1, xr0, xr1, ee, esrc, edst, att):
    mesh = plsc.VectorSubcoreMesh(core_axis_name="c", subcore_axis_name="s")
    f = functools.partial(
        pl.kernel,
        out_type=[
            jax.ShapeDtypeStruct((E,), jnp.float32),
            jax.ShapeDtypeStruct((NT2, 16), jnp.float32),
        ],
        mesh=mesh,
        scratch_types=(
            [pltpu.VMEM((K2B,), jnp.int32)] * 4
            + [pltpu.VMEM((K2B, DH), jnp.float32)] * 8
            + [pltpu.VMEM((K2B, D), jnp.float32)] * 2
            + [pltpu.VMEM((48,), jnp.float32)] * 2
            + [pltpu.VMEM((D,), jnp.float32),
               pltpu.VMEM((16,), jnp.float32),
               pltpu.VMEM((D,), jnp.float32)]
            + [pltpu.SemaphoreType.DMA] * 8
        ),
        compiler_params=pltpu.CompilerParams(needs_layout_passes=False),
    )(_k2_body)
    return f(xl0, xl1, xr0, xr1, ee, esrc, edst, att)


def _k3_body(xl0, xl1, esrc, edst, logits, tmax, mself,
             accout, sparts,
             acc, srcbA, srcbB, dstbA, dstbB, dstsA, dstsB,
             gbA, gbB, stageA, stageB,
             lbA, lbB, ebA, ebB, tb, msb, stab,
             s_idx, s_gb, s_lg, s_scA, s_scB):
    c = lax.axis_index("c")
    s = lax.axis_index("s")
    iota = lax.broadcasted_iota(jnp.int32, (16,), 0)
    srcb = (srcbA, srcbB)
    dstb = (dstbA, dstbB)
    dsts = (dstsA, dstsB)
    gbb = (gbA, gbB)
    stage = (stageA, stageB)
    lbb = (lbA, lbB)
    ebb = (ebA, ebB)
    s_sc = (s_scA, s_scB)

    pltpu.sync_copy(tmax, tb)
    pltpu.sync_copy(mself.at[0, pl.ds(0, 16)], msb)
    m16 = msb[...]
    for r in range(NT2):
        m16 = jnp.maximum(m16, tb[r])
    for sh in (8, 4, 2, 1):
        msb[...] = m16
        m16 = jnp.maximum(m16, plsc.load_gather(msb, [(iota + sh) & 15]))

    z16 = jnp.zeros((16,), jnp.float32)

    def zrow(i, _):
        for k in range(DH // 16):
            stageA[i, pl.ds(16 * k, 16)] = z16
        return 0

    lax.fori_loop(0, K3B, zrow, 0)

    def zacc(q, _):
        pltpu.sync_copy(stageA, acc.at[pl.ds(s * 640 + q * K3B, K3B)])
        return 0

    lax.fori_loop(0, 640 // K3B, zacc, 0)

    @pl.when(c == 0)
    def _():
        def zs(i, _):
            stab[pl.ds(16 * i, 16)] = z16
            return 0

        lax.fori_loop(0, NP // 16, zs, 0)

    plsc.subcore_barrier()

    ebase = s * EPT3

    def run(xtab, do_s):
        def idx_descs(g, p):
            e0 = ebase + g * K3B
            return (pltpu.make_async_copy(esrc.at[pl.ds(e0, K3B)],
                                          srcb[p], s_idx),
                    pltpu.make_async_copy(edst.at[pl.ds(e0, K3B)],
                                          dstb[p], s_idx))

        def gather_descs(g, p):
            e0 = ebase + g * K3B
            return (pltpu.make_async_copy(xtab.at[srcb[p]], gbb[p], s_gb),
                    pltpu.make_async_copy(logits.at[pl.ds(e0, K3B)],
                                          lbb[p], s_lg))

        def scat_desc(p):
            return pltpu.make_async_copy(stage[p], acc.at[dsts[p]], s_sc[p])

        def scat_start(p):
            pltpu.async_copy(stage[p], acc.at[dsts[p]], s_sc[p], add=True)

        def compute(g, p):
            lb, gb, stg = lbb[p], gbb[p], stage[p]
            for off in (0, 16, 24):
                dsts[p][pl.ds(off, 16)] = dstb[p][pl.ds(off, 16)]
            exb = ebb[p]
            for off in (0, 16, 24):
                exb[pl.ds(off, 16)] = jnp.exp(lb[pl.ds(off, 16)] - m16)

            if do_s:
                for off, lo in ((0, 0), (16, 0), (24, 8)):
                    dst16 = dstb[p][pl.ds(off, 16)]
                    ex16 = exb[pl.ds(off, 16)]
                    for r in range(lo, 16):
                        plsc.addupdate_scatter(stab, [dst16], ex16,
                                               mask=iota == r)

            def edge(j, _):
                exs = plsc.load_gather(exb, [jnp.full((16,), j, jnp.int32)])
                for k in range(DH // 16):
                    stg[j, pl.ds(16 * k, 16)] = gb[j, pl.ds(16 * k, 16)] * exs
                return 0

            lax.fori_loop(0, K3B, edge, 0)

        def body_half(g, p):
            for d in gather_descs(g, p):
                d.wait()
            for d in idx_descs(g + 1, 1 - p):
                d.wait()
            for d in gather_descs(g + 1, 1 - p):
                d.start()

            @pl.when(g >= 2)
            def _():
                scat_desc(p).wait()

            compute(g, p)
            scat_start(p)
            gnxt = jnp.minimum(g + 2, NB3 - 1)
            for d in idx_descs(gnxt, p):
                d.start()

        for d in idx_descs(0, 0):
            d.start()
        for d in idx_descs(0, 0):
            d.wait()
        for d in gather_descs(0, 0):
            d.start()
        for d in idx_descs(1, 1):
            d.start()

        def pair(gp, _):
            g = gp * 2
            body_half(g, 0)
            body_half(g + 1, 1)
            return 0

        lax.fori_loop(0, (NB3 - 2) // 2, pair, 0)

        body_half(jnp.int32(NB3 - 2), 0)
        gl = NB3 - 1
        for d in gather_descs(gl, 1):
            d.wait()
        scat_desc(1).wait()
        compute(jnp.int32(gl), 1)
        scat_start(1)
        for d in idx_descs(gl, 0):
            d.wait()
        scat_desc(0).wait()
        scat_desc(1).wait()

    @pl.when(c == 0)
    def _():
        run(xl0, True)

    @pl.when(c == 1)
    def _():
        run(xl1, False)

    plsc.subcore_barrier()
    pltpu.sync_copy(acc.at[pl.ds(s * 640, 640)],
                    accout.at[pl.ds(c * NP + s * 640, 640)])

    @pl.when(c == 0)
    def _():
        pltpu.sync_copy(stab, sparts.at[pl.ds(s * NP, NP)])


def _k3(xl0, xl1, esrc, edst, logits, tmax, mself):
    mesh = plsc.VectorSubcoreMesh(core_axis_name="c", subcore_axis_name="s")
    f = functools.partial(
        pl.kernel,
        out_type=[
            jax.ShapeDtypeStruct((2 * NP, DH), jnp.float32),
            jax.ShapeDtypeStruct((16 * NP,), jnp.float32),
        ],
        mesh=mesh,
        scratch_types=(
            [pltpu.VMEM_SHARED((NP, DH), jnp.float32)]
            + [pltpu.VMEM((K3B,), jnp.int32)] * 6
            + [pltpu.VMEM((K3B, DH), jnp.float32)] * 4
            + [pltpu.VMEM((K3B,), jnp.float32)] * 4
            + [pltpu.VMEM((NT2, 16), jnp.float32),
               pltpu.VMEM((16,), jnp.float32),
               pltpu.VMEM((NP,), jnp.float32)]
            + [pltpu.SemaphoreType.DMA] * 5
        ),
        compiler_params=pltpu.CompilerParams(needs_layout_passes=False),
    )(_k3_body)
    return f(xl0, xl1, esrc, edst, logits, tmax, mself)


def _k4_body(a0_ref, a1_ref, sp_ref, xl0_ref, xl1_ref, xr0_ref, xr1_ref,
             tmax_ref, m_ref, sum_ref, we_ref, att_ref, bias_ref, out_ref):
    M = jnp.maximum(jnp.max(tmax_ref[...]), jnp.max(m_ref[...]))
    crow = jnp.dot(sum_ref[...] * (1.0 / E), we_ref[...],
                   preferred_element_type=jnp.float32)
    xl = jnp.concatenate([xl0_ref[...], xl1_ref[...]], axis=1)
    xr = jnp.concatenate([xr0_ref[...], xr1_ref[...]], axis=1)
    z = xl + xr + crow
    z = jnp.maximum(z, NEG * z)
    ls = jnp.sum(z * att_ref[...], axis=1, keepdims=True)
    exs = jnp.exp(ls - M)
    ssum = jnp.sum(sp_ref[...], axis=0)[:, None] + exs
    num = jnp.concatenate([a0_ref[...], a1_ref[...]], axis=1) + exs * xl
    out_ref[...] = jnp.maximum(num / ssum + bias_ref[...], 0.0)


def _k4(accout, sparts2, xl0, xl1, xr0, xr1, tmax, mself, sumea, W_e,
        att2, bias2):
    return pl.pallas_call(
        _k4_body,
        grid=(NP // NBLK,),
        in_specs=[
            pl.BlockSpec((NBLK, DH), lambda i: (i, 0)),
            pl.BlockSpec((NBLK, DH), lambda i: (i + NP // NBLK, 0)),
            pl.BlockSpec((16, NBLK), lambda i: (0, i)),
            pl.BlockSpec((NBLK, DH), lambda i: (i, 0)),
            pl.BlockSpec((NBLK, DH), lambda i: (i, 0)),
            pl.BlockSpec((NBLK, DH), lambda i: (i, 0)),
            pl.BlockSpec((NBLK, DH), lambda i: (i, 0)),
            pl.BlockSpec((NT2, 16), lambda i: (0, 0)),
            pl.BlockSpec((8, 128), lambda i: (0, 0)),
            pl.BlockSpec((1, DE), lambda i: (0, 0)),
            pl.BlockSpec((DE, D), lambda i: (0, 0)),
            pl.BlockSpec((1, D), lambda i: (0, 0)),
            pl.BlockSpec((1, D), lambda i: (0, 0)),
        ],
        out_specs=pl.BlockSpec((NBLK, D), lambda i: (i, 0)),
        out_shape=jax.ShapeDtypeStruct((NP, D), jnp.float32),
    )(accout, accout, sparts2, xl0, xl1, xr0, xr1, tmax, mself, sumea, W_e,
      att2, bias2)


def kernel(x, edge_index, edge_attr, return_attention_weights,
           W_l, W_r, W_e, att, bias):
    ei = edge_index.astype(jnp.int32)
    esrc = ei[0]
    edst = ei[1]
    att2 = att.reshape(1, D)
    bias2 = bias.reshape(1, D)
    xp = jnp.zeros((NP, D), jnp.float32).at[:N].set(x)
    ee, sumea = _kee(edge_attr, W_e)
    xl0, xl1, xr0, xr1, mself = _k1(xp, W_l, W_r, W_e, sumea, att2)
    logits, tmax = _k2(xl0, xl1, xr0, xr1, ee, esrc, edst, att)
    accout, sparts = _k3(xl0, xl1, esrc, edst, logits, tmax, mself)
    sparts2 = sparts.reshape(16, NP)
    out = _k4(accout, sparts2, xl0, xl1, xr0, xr1, tmax, mself, sumea, W_e,
              att2, bias2)
    return out[:N]


# speedup vs baseline: 4.5177x; 1.0501x over previous
import functools

import jax
import jax.numpy as jnp
from jax import lax
from jax.experimental import pallas as pl
from jax.experimental.pallas import tpu as pltpu
from jax.experimental.pallas import tpu_sc as plsc

N = 10000
E = 160000
D = 256
DH = 128
DE = 16
NEG = 0.2
NEG_INF = -3e38
NP = 10240
NBLK = 512

NT2 = 32
EPT2 = E // NT2
K2B = 40
NB2 = EPT2 // K2B

EPT3 = E // 16
K3B = 80
NB3 = EPT3 // K3B


def _kee_body(ea_ref, we_ref, ee_ref, sum_ref):
    i = pl.program_id(0)
    ea = ea_ref[...]
    ee_ref[...] = jnp.dot(ea, we_ref[...], preferred_element_type=jnp.float32)

    @pl.when(i == 0)
    def _():
        sum_ref[...] = jnp.zeros((1, DE), jnp.float32)

    sum_ref[...] += jnp.sum(ea, axis=0, keepdims=True)


def _kee(edge_attr, W_e):
    blk = 2000
    return pl.pallas_call(
        _kee_body,
        grid=(E // blk,),
        in_specs=[
            pl.BlockSpec((blk, DE), lambda i: (i, 0)),
            pl.BlockSpec((DE, D), lambda i: (0, 0)),
        ],
        out_specs=[
            pl.BlockSpec((blk, D), lambda i: (i, 0)),
            pl.BlockSpec((1, DE), lambda i: (0, 0)),
        ],
        out_shape=[
            jax.ShapeDtypeStruct((E, D), jnp.float32),
            jax.ShapeDtypeStruct((1, DE), jnp.float32),
        ],
    )(edge_attr, W_e)


def _k1_body(x_ref, wl_ref, wr_ref, we_ref, sum_ref, att_ref,
             xl0_ref, xl1_ref, xr0_ref, xr1_ref, m_ref):
    i = pl.program_id(0)
    xv = x_ref[...]
    xl = jnp.dot(xv, wl_ref[...], preferred_element_type=jnp.float32)
    xr = jnp.dot(xv, wr_ref[...], preferred_element_type=jnp.float32)
    xl0_ref[...] = xl[:, :DH]
    xl1_ref[...] = xl[:, DH:]
    xr0_ref[...] = xr[:, :DH]
    xr1_ref[...] = xr[:, DH:]
    crow = jnp.dot(sum_ref[...] * (1.0 / E), we_ref[...],
                   preferred_element_type=jnp.float32)
    z = xl + xr + crow
    z = jnp.maximum(z, NEG * z)
    ls = jnp.sum(z * att_ref[...], axis=1)
    m = jnp.max(ls)

    @pl.when(i == 0)
    def _():
        m_ref[...] = jnp.full((8, 128), NEG_INF, jnp.float32)

    m_ref[...] = jnp.maximum(m_ref[...], m)


def _k1(xp, W_l, W_r, W_e, sumea, att2):
    return pl.pallas_call(
        _k1_body,
        grid=(NP // NBLK,),
        in_specs=[
            pl.BlockSpec((NBLK, D), lambda i: (i, 0)),
            pl.BlockSpec((D, D), lambda i: (0, 0)),
            pl.BlockSpec((D, D), lambda i: (0, 0)),
            pl.BlockSpec((DE, D), lambda i: (0, 0)),
            pl.BlockSpec((1, DE), lambda i: (0, 0)),
            pl.BlockSpec((1, D), lambda i: (0, 0)),
        ],
        out_specs=[
            pl.BlockSpec((NBLK, DH), lambda i: (i, 0)),
            pl.BlockSpec((NBLK, DH), lambda i: (i, 0)),
            pl.BlockSpec((NBLK, DH), lambda i: (i, 0)),
            pl.BlockSpec((NBLK, DH), lambda i: (i, 0)),
            pl.BlockSpec((8, 128), lambda i: (0, 0)),
        ],
        out_shape=[
            jax.ShapeDtypeStruct((NP, DH), jnp.float32),
            jax.ShapeDtypeStruct((NP, DH), jnp.float32),
            jax.ShapeDtypeStruct((NP, DH), jnp.float32),
            jax.ShapeDtypeStruct((NP, DH), jnp.float32),
            jax.ShapeDtypeStruct((8, 128), jnp.float32),
        ],
    )(xp, W_l, W_r, W_e, sumea, att2)


def _k2_body(xl0, xl1, xr0, xr1, ee, esrc, edst, att,
             logits, tmax,
             srcbA, srcbB, dstbA, dstbB,
             bxl0A, bxl0B, bxl1A, bxl1B, bxr0A, bxr0B, bxr1A, bxr1B,
             beeA, beeB, lbufA, lbufB, attv, mxb, tbuf,
             s_idx, s_g0, s_g1, s_g2, s_g3, s_ee, s_logA, s_logB):
    c = lax.axis_index("c")
    s = lax.axis_index("s")
    wid = s * 2 + c
    base = wid * EPT2
    srcb = (srcbA, srcbB)
    dstb = (dstbA, dstbB)
    bxl0b = (bxl0A, bxl0B)
    bxl1b = (bxl1A, bxl1B)
    bxr0b = (bxr0A, bxr0B)
    bxr1b = (bxr1A, bxr1B)
    beeb = (beeA, beeB)
    lbufb = (lbufA, lbufB)

    pltpu.sync_copy(att, attv)
    attc = [attv[pl.ds(16 * k, 16)] for k in range(16)]
    iota = lax.broadcasted_iota(jnp.int32, (16,), 0)

    def idx_descs(g, p):
        return (pltpu.make_async_copy(esrc.at[pl.ds(base + g * K2B, K2B)],
                                      srcb[p], s_idx),
                pltpu.make_async_copy(edst.at[pl.ds(base + g * K2B, K2B)],
                                      dstb[p], s_idx))

    def gather_descs(g, p):
        e0 = base + g * K2B
        return (pltpu.make_async_copy(xl0.at[srcb[p]], bxl0b[p], s_g0),
                pltpu.make_async_copy(xl1.at[srcb[p]], bxl1b[p], s_g1),
                pltpu.make_async_copy(xr0.at[dstb[p]], bxr0b[p], s_g2),
                pltpu.make_async_copy(xr1.at[dstb[p]], bxr1b[p], s_g3),
                pltpu.make_async_copy(ee.at[pl.ds(e0, K2B)], beeb[p], s_ee))

    def log_desc(g, p):
        e0 = base + g * K2B
        return pltpu.make_async_copy(lbufb[p].at[pl.ds(0, K2B)],
                                     logits.at[pl.ds(e0, K2B)],
                                     s_logA if p == 0 else s_logB)

    def compute(g, p, mx16):
        bl0, bl1, br0, br1, be, lb = (bxl0b[p], bxl1b[p], bxr0b[p],
                                      bxr1b[p], beeb[p], lbufb[p])
        for gg in range(3):
            ne = 16 if gg < 2 else K2B - 32

            def edge(je, _):
                j = gg * 16 + je
                t = jnp.zeros((16,), jnp.float32)
                for k in range(8):
                    z = (bl0[j, pl.ds(16 * k, 16)]
                         + br0[j, pl.ds(16 * k, 16)]
                         + be[j, pl.ds(16 * k, 16)])
                    z = jnp.maximum(z, NEG * z)
                    t = t + attc[k] * z
                for k in range(8):
                    z = (bl1[j, pl.ds(16 * k, 16)]
                         + br1[j, pl.ds(16 * k, 16)]
                         + be[j, pl.ds(DH + 16 * k, 16)])
                    z = jnp.maximum(z, NEG * z)
                    t = t + attc[8 + k] * z
                plsc.store_scatter(tbuf, [iota * 16 + je], t)
                return 0

            lax.fori_loop(0, ne, edge, 0)
            colsum = tbuf[pl.ds(0, 16)]
            for r in range(1, 16):
                colsum = colsum + tbuf[pl.ds(16 * r, 16)]
            lb[pl.ds(16 * gg, 16)] = colsum
            mx16 = jnp.maximum(mx16, colsum)
        return mx16

    def body_half(g, p, mx16):
        for d in gather_descs(g, p):
            d.wait()
        for d in idx_descs(g + 1, 1 - p):
            d.wait()
        for d in gather_descs(g + 1, 1 - p):
            d.start()
        gnxt = jnp.minimum(g + 2, NB2 - 1)
        for d in idx_descs(gnxt, p):
            d.start()

        @pl.when(g >= 2)
        def _():
            log_desc(g - 2, p).wait()

        mx16 = compute(g, p, mx16)
        log_desc(g, p).start()
        return mx16

    for d in idx_descs(0, 0):
        d.start()
    for d in idx_descs(0, 0):
        d.wait()
    for d in gather_descs(0, 0):
        d.start()
    for d in idx_descs(1, 1):
        d.start()

    def pair(gp, mx16):
        g = gp * 2
        mx16 = body_half(g, 0, mx16)
        mx16 = body_half(g + 1, 1, mx16)
        return mx16

    mx16 = lax.fori_loop(0, (NB2 - 1) // 2, pair,
                         jnp.full((16,), NEG_INF, jnp.float32))

    gl = NB2 - 1
    for d in gather_descs(gl, 0):
        d.wait()
    log_desc(gl - 2, 0).wait()
    mx16 = compute(gl, 0, mx16)
    log_desc(gl, 0).start()
    for d in idx_descs(gl, 1):
        d.wait()
    log_desc(gl - 1, 1).wait()
    log_desc(gl, 0).wait()

    mxb[...] = mx16
    pltpu.sync_copy(mxb, tmax.at[wid])


def _k2(xl0, xl1, xr0, xr1, ee, esrc, edst, att):
    mesh = plsc.VectorSubcoreMesh(core_axis_name="c", subcore_axis_name="s")
    f = functools.partial(
        pl.kernel,
        out_type=[
            jax.ShapeDtypeStruct((E,), jnp.float32),
            jax.ShapeDtypeStruct((NT2, 16), jnp.float32),
        ],
        mesh=mesh,
        scratch_types=(
            [pltpu.VMEM((K2B,), jnp.int32)] * 4
            + [pltpu.VMEM((K2B, DH), jnp.float32)] * 8
            + [pltpu.VMEM((K2B, D), jnp.float32)] * 2
            + [pltpu.VMEM((48,), jnp.float32)] * 2
            + [pltpu.VMEM((D,), jnp.float32),
               pltpu.VMEM((16,), jnp.float32),
               pltpu.VMEM((D,), jnp.float32)]
            + [pltpu.SemaphoreType.DMA] * 8
        ),
        compiler_params=pltpu.CompilerParams(needs_layout_passes=False),
    )(_k2_body)
    return f(xl0, xl1, xr0, xr1, ee, esrc, edst, att)


def _k25_body(edst, logits, tmax, mself,
              sparts,
              dstb, lb, stab, tb, msb):
    c = lax.axis_index("c")
    s = lax.axis_index("s")
    wid = s * 2 + c
    base = wid * EPT2
    iota = lax.broadcasted_iota(jnp.int32, (16,), 0)

    pltpu.sync_copy(tmax, tb)
    pltpu.sync_copy(mself.at[0, pl.ds(0, 16)], msb)
    m16 = msb[...]
    for r in range(NT2):
        m16 = jnp.maximum(m16, tb[r])
    for sh in (8, 4, 2, 1):
        msb[...] = m16
        m16 = jnp.maximum(m16, plsc.load_gather(msb, [(iota + sh) & 15]))

    z16 = jnp.zeros((16,), jnp.float32)

    def zs(i, _):
        stab[pl.ds(16 * i, 16)] = z16
        return 0

    lax.fori_loop(0, NP // 16, zs, 0)

    pltpu.sync_copy(edst.at[pl.ds(base, EPT2)], dstb)
    pltpu.sync_copy(logits.at[pl.ds(base, EPT2)], lb)

    def grp(gi, _):
        off = gi * 16
        dst16 = dstb[pl.ds(off, 16)]
        ex16 = jnp.exp(lb[pl.ds(off, 16)] - m16)
        for r in range(16):
            plsc.addupdate_scatter(stab, [dst16], ex16, mask=iota == r)
        return 0

    lax.fori_loop(0, EPT2 // 16 , grp, 0)
    dst16 = dstb[pl.ds(EPT2 - 16, 16)]
    ex16 = jnp.exp(lb[pl.ds(EPT2 - 16, 16)] - m16)
    for r in range(8, 16):
        plsc.addupdate_scatter(stab, [dst16], ex16, mask=iota == r)

    pltpu.sync_copy(stab, sparts.at[pl.ds(wid * NP, NP)])


def _k25(edst, logits, tmax, mself):
    mesh = plsc.VectorSubcoreMesh(core_axis_name="c", subcore_axis_name="s")
    f = functools.partial(
        pl.kernel,
        out_type=[jax.ShapeDtypeStruct((NT2 * NP,), jnp.float32)],
        mesh=mesh,
        scratch_types=(
            [pltpu.VMEM((EPT2,), jnp.int32),
             pltpu.VMEM((EPT2,), jnp.float32),
             pltpu.VMEM((NP,), jnp.float32),
             pltpu.VMEM((NT2, 16), jnp.float32),
             pltpu.VMEM((16,), jnp.float32)]
        ),
        compiler_params=pltpu.CompilerParams(needs_layout_passes=False),
    )(_k25_body)
    return f(edst, logits, tmax, mself)[0]


def _k3_body(xl0, xl1, esrc, edst, logits, tmax, mself,
             accout,
             acc, srcbA, srcbB, dstbA, dstbB, dstsA, dstsB,
             gbA, gbB, stageA, stageB,
             lbA, lbB, ebA, ebB, tb, msb,
             s_idx, s_gb, s_lg, s_scA, s_scB):
    c = lax.axis_index("c")
    s = lax.axis_index("s")
    iota = lax.broadcasted_iota(jnp.int32, (16,), 0)
    srcb = (srcbA, srcbB)
    dstb = (dstbA, dstbB)
    dsts = (dstsA, dstsB)
    gbb = (gbA, gbB)
    stage = (stageA, stageB)
    lbb = (lbA, lbB)
    ebb = (ebA, ebB)
    s_sc = (s_scA, s_scB)

    pltpu.sync_copy(tmax, tb)
    pltpu.sync_copy(mself.at[0, pl.ds(0, 16)], msb)
    m16 = msb[...]
    for r in range(NT2):
        m16 = jnp.maximum(m16, tb[r])
    for sh in (8, 4, 2, 1):
        msb[...] = m16
        m16 = jnp.maximum(m16, plsc.load_gather(msb, [(iota + sh) & 15]))

    z16 = jnp.zeros((16,), jnp.float32)

    def zrow(i, _):
        for k in range(DH // 16):
            stageA[i, pl.ds(16 * k, 16)] = z16
        return 0

    lax.fori_loop(0, K3B, zrow, 0)

    def zacc(q, _):
        pltpu.sync_copy(stageA, acc.at[pl.ds(s * 640 + q * K3B, K3B)])
        return 0

    lax.fori_loop(0, 640 // K3B, zacc, 0)
    plsc.subcore_barrier()

    ebase = s * EPT3

    def run(xtab):
        def idx_descs(g, p):
            e0 = ebase + g * K3B
            return (pltpu.make_async_copy(esrc.at[pl.ds(e0, K3B)],
                                          srcb[p], s_idx),
                    pltpu.make_async_copy(edst.at[pl.ds(e0, K3B)],
                                          dstb[p], s_idx))

        def gather_descs(g, p):
            e0 = ebase + g * K3B
            return (pltpu.make_async_copy(xtab.at[srcb[p]], gbb[p], s_gb),
                    pltpu.make_async_copy(logits.at[pl.ds(e0, K3B)],
                                          lbb[p], s_lg))

        def scat_desc(p):
            return pltpu.make_async_copy(stage[p], acc.at[dsts[p]], s_sc[p])

        def scat_start(p):
            pltpu.async_copy(stage[p], acc.at[dsts[p]], s_sc[p], add=True)

        def compute(g, p):
            lb, gb, stg = lbb[p], gbb[p], stage[p]
            exb = ebb[p]
            for off in range(0, K3B, 16):
                dsts[p][pl.ds(off, 16)] = dstb[p][pl.ds(off, 16)]
                exb[pl.ds(off, 16)] = jnp.exp(lb[pl.ds(off, 16)] - m16)

            def edge(j, _):
                exs = plsc.load_gather(exb, [jnp.full((16,), j, jnp.int32)])
                for k in range(DH // 16):
                    stg[j, pl.ds(16 * k, 16)] = gb[j, pl.ds(16 * k, 16)] * exs
                return 0

            lax.fori_loop(0, K3B, edge, 0)

        def body_half(g, p):
            for d in gather_descs(g, p):
                d.wait()
            for d in idx_descs(g + 1, 1 - p):
                d.wait()
            for d in gather_descs(g + 1, 1 - p):
                d.start()

            @pl.when(g >= 2)
            def _():
                scat_desc(p).wait()

            compute(g, p)
            scat_start(p)
            gnxt = jnp.minimum(g + 2, NB3 - 1)
            for d in idx_descs(gnxt, p):
                d.start()

        for d in idx_descs(0, 0):
            d.start()
        for d in idx_descs(0, 0):
            d.wait()
        for d in gather_descs(0, 0):
            d.start()
        for d in idx_descs(1, 1):
            d.start()

        def pair(gp, _):
            g = gp * 2
            body_half(g, 0)
            body_half(g + 1, 1)
            return 0

        lax.fori_loop(0, (NB3 - 1) // 2, pair, 0)

        gl = NB3 - 1
        for d in gather_descs(gl, 0):
            d.wait()
        scat_desc(0).wait()
        compute(jnp.int32(gl), 0)
        scat_start(0)
        for d in idx_descs(gl, 1):
            d.wait()
        scat_desc(1).wait()
        scat_desc(0).wait()

    @pl.when(c == 0)
    def _():
        run(xl0)

    @pl.when(c == 1)
    def _():
        run(xl1)

    plsc.subcore_barrier()
    pltpu.sync_copy(acc.at[pl.ds(s * 640, 640)],
                    accout.at[pl.ds(c * NP + s * 640, 640)])


def _k3(xl0, xl1, esrc, edst, logits, tmax, mself):
    mesh = plsc.VectorSubcoreMesh(core_axis_name="c", subcore_axis_name="s")
    f = functools.partial(
        pl.kernel,
        out_type=[
            jax.ShapeDtypeStruct((2 * NP, DH), jnp.float32),
        ],
        mesh=mesh,
        scratch_types=(
            [pltpu.VMEM_SHARED((NP, DH), jnp.float32)]
            + [pltpu.VMEM((K3B,), jnp.int32)] * 6
            + [pltpu.VMEM((K3B, DH), jnp.float32)] * 4
            + [pltpu.VMEM((K3B,), jnp.float32)] * 4
            + [pltpu.VMEM((NT2, 16), jnp.float32),
               pltpu.VMEM((16,), jnp.float32)]
            + [pltpu.SemaphoreType.DMA] * 5
        ),
        compiler_params=pltpu.CompilerParams(needs_layout_passes=False),
    )(_k3_body)
    return f(xl0, xl1, esrc, edst, logits, tmax, mself)[0]


def _k4_body(a0_ref, a1_ref, sp_ref, xl0_ref, xl1_ref, xr0_ref, xr1_ref,
             tmax_ref, m_ref, sum_ref, we_ref, att_ref, bias_ref, out_ref):
    M = jnp.maximum(jnp.max(tmax_ref[...]), jnp.max(m_ref[...]))
    crow = jnp.dot(sum_ref[...] * (1.0 / E), we_ref[...],
                   preferred_element_type=jnp.float32)
    xl = jnp.concatenate([xl0_ref[...], xl1_ref[...]], axis=1)
    xr = jnp.concatenate([xr0_ref[...], xr1_ref[...]], axis=1)
    z = xl + xr + crow
    z = jnp.maximum(z, NEG * z)
    ls = jnp.sum(z * att_ref[...], axis=1, keepdims=True)
    exs = jnp.exp(ls - M)
    ssum = jnp.sum(sp_ref[...], axis=0)[:, None] + exs
    num = jnp.concatenate([a0_ref[...], a1_ref[...]], axis=1) + exs * xl
    out_ref[...] = jnp.maximum(num / ssum + bias_ref[...], 0.0)


def _k4(accout, sparts2, xl0, xl1, xr0, xr1, tmax, mself, sumea, W_e,
        att2, bias2):
    return pl.pallas_call(
        _k4_body,
        grid=(NP // NBLK,),
        in_specs=[
            pl.BlockSpec((NBLK, DH), lambda i: (i, 0)),
            pl.BlockSpec((NBLK, DH), lambda i: (i + NP // NBLK, 0)),
            pl.BlockSpec((NT2, NBLK), lambda i: (0, i)),
            pl.BlockSpec((NBLK, DH), lambda i: (i, 0)),
            pl.BlockSpec((NBLK, DH), lambda i: (i, 0)),
            pl.BlockSpec((NBLK, DH), lambda i: (i, 0)),
            pl.BlockSpec((NBLK, DH), lambda i: (i, 0)),
            pl.BlockSpec((NT2, 16), lambda i: (0, 0)),
            pl.BlockSpec((8, 128), lambda i: (0, 0)),
            pl.BlockSpec((1, DE), lambda i: (0, 0)),
            pl.BlockSpec((DE, D), lambda i: (0, 0)),
            pl.BlockSpec((1, D), lambda i: (0, 0)),
            pl.BlockSpec((1, D), lambda i: (0, 0)),
        ],
        out_specs=pl.BlockSpec((NBLK, D), lambda i: (i, 0)),
        out_shape=jax.ShapeDtypeStruct((NP, D), jnp.float32),
    )(accout, accout, sparts2, xl0, xl1, xr0, xr1, tmax, mself, sumea, W_e,
      att2, bias2)


def kernel(x, edge_index, edge_attr, return_attention_weights,
           W_l, W_r, W_e, att, bias):
    ei = edge_index.astype(jnp.int32)
    esrc = ei[0]
    edst = ei[1]
    att2 = att.reshape(1, D)
    bias2 = bias.reshape(1, D)
    xp = jnp.zeros((NP, D), jnp.float32).at[:N].set(x)
    ee, sumea = _kee(edge_attr, W_e)
    xl0, xl1, xr0, xr1, mself = _k1(xp, W_l, W_r, W_e, sumea, att2)
    logits, tmax = _k2(xl0, xl1, xr0, xr1, ee, esrc, edst, att)
    sparts = _k25(edst, logits, tmax, mself)
    accout = _k3(xl0, xl1, esrc, edst, logits, tmax, mself)
    sparts2 = sparts.reshape(NT2, NP)
    out = _k4(accout, sparts2, xl0, xl1, xr0, xr1, tmax, mself, sumea, W_e,
              att2, bias2)
    return out[:N]


# speedup vs baseline: 4.5339x; 1.0036x over previous
import functools

import jax
import jax.numpy as jnp
from jax import lax
from jax.experimental import pallas as pl
from jax.experimental.pallas import tpu as pltpu
from jax.experimental.pallas import tpu_sc as plsc

N = 10000
E = 160000
D = 256
DH = 128
DE = 16
NEG = 0.2
NEG_INF = -3e38
NP = 10240
NBLK = 512

NT2 = 32
EPT2 = E // NT2
K2B = 40
NB2 = EPT2 // K2B

EPT3 = E // 16
K3B = 80
NB3 = EPT3 // K3B


def _kee_body(ea_ref, we_ref, ee_ref, sum_ref):
    i = pl.program_id(0)
    ea = ea_ref[...]
    ee_ref[...] = jnp.dot(ea, we_ref[...], preferred_element_type=jnp.float32)

    @pl.when(i == 0)
    def _():
        sum_ref[...] = jnp.zeros((1, DE), jnp.float32)

    sum_ref[...] += jnp.sum(ea, axis=0, keepdims=True)


def _kee(edge_attr, W_e):
    blk = 2000
    return pl.pallas_call(
        _kee_body,
        grid=(E // blk,),
        in_specs=[
            pl.BlockSpec((blk, DE), lambda i: (i, 0)),
            pl.BlockSpec((DE, D), lambda i: (0, 0)),
        ],
        out_specs=[
            pl.BlockSpec((blk, D), lambda i: (i, 0)),
            pl.BlockSpec((1, DE), lambda i: (0, 0)),
        ],
        out_shape=[
            jax.ShapeDtypeStruct((E, D), jnp.float32),
            jax.ShapeDtypeStruct((1, DE), jnp.float32),
        ],
    )(edge_attr, W_e)


def _k1_body(x_ref, wl_ref, wr_ref, we_ref, sum_ref, att_ref,
             xl0_ref, xl1_ref, xr0_ref, xr1_ref, m_ref):
    i = pl.program_id(0)
    xv = x_ref[...]
    xl = jnp.dot(xv, wl_ref[...], preferred_element_type=jnp.float32)
    xr = jnp.dot(xv, wr_ref[...], preferred_element_type=jnp.float32)
    xl0_ref[...] = xl[:, :DH]
    xl1_ref[...] = xl[:, DH:]
    xr0_ref[...] = xr[:, :DH]
    xr1_ref[...] = xr[:, DH:]
    crow = jnp.dot(sum_ref[...] * (1.0 / E), we_ref[...],
                   preferred_element_type=jnp.float32)
    z = xl + xr + crow
    z = jnp.maximum(z, NEG * z)
    ls = jnp.sum(z * att_ref[...], axis=1)
    m = jnp.max(ls)

    @pl.when(i == 0)
    def _():
        m_ref[...] = jnp.full((8, 128), NEG_INF, jnp.float32)

    m_ref[...] = jnp.maximum(m_ref[...], m)


def _k1(xp, W_l, W_r, W_e, sumea, att2):
    return pl.pallas_call(
        _k1_body,
        grid=(NP // NBLK,),
        in_specs=[
            pl.BlockSpec((NBLK, D), lambda i: (i, 0)),
            pl.BlockSpec((D, D), lambda i: (0, 0)),
            pl.BlockSpec((D, D), lambda i: (0, 0)),
            pl.BlockSpec((DE, D), lambda i: (0, 0)),
            pl.BlockSpec((1, DE), lambda i: (0, 0)),
            pl.BlockSpec((1, D), lambda i: (0, 0)),
        ],
        out_specs=[
            pl.BlockSpec((NBLK, DH), lambda i: (i, 0)),
            pl.BlockSpec((NBLK, DH), lambda i: (i, 0)),
            pl.BlockSpec((NBLK, DH), lambda i: (i, 0)),
            pl.BlockSpec((NBLK, DH), lambda i: (i, 0)),
            pl.BlockSpec((8, 128), lambda i: (0, 0)),
        ],
        out_shape=[
            jax.ShapeDtypeStruct((NP, DH), jnp.float32),
            jax.ShapeDtypeStruct((NP, DH), jnp.float32),
            jax.ShapeDtypeStruct((NP, DH), jnp.float32),
            jax.ShapeDtypeStruct((NP, DH), jnp.float32),
            jax.ShapeDtypeStruct((8, 128), jnp.float32),
        ],
    )(xp, W_l, W_r, W_e, sumea, att2)


def _k2_body(xl0, xl1, xr0, xr1, ee, esrc, edst, att,
             logits, tmax,
             srcbA, srcbB, dstbA, dstbB,
             bxl0A, bxl0B, bxl1A, bxl1B, bxr0A, bxr0B, bxr1A, bxr1B,
             beeA, beeB, lbufA, lbufB, attv, mxb, tbuf,
             s_idx, s_g0, s_g1, s_g2, s_g3, s_ee, s_logA, s_logB):
    c = lax.axis_index("c")
    s = lax.axis_index("s")
    wid = s * 2 + c
    base = wid * EPT2
    srcb = (srcbA, srcbB)
    dstb = (dstbA, dstbB)
    bxl0b = (bxl0A, bxl0B)
    bxl1b = (bxl1A, bxl1B)
    bxr0b = (bxr0A, bxr0B)
    bxr1b = (bxr1A, bxr1B)
    beeb = (beeA, beeB)
    lbufb = (lbufA, lbufB)

    pltpu.sync_copy(att, attv)
    attc = [attv[pl.ds(16 * k, 16)] for k in range(16)]
    iota = lax.broadcasted_iota(jnp.int32, (16,), 0)

    def idx_descs(g, p):
        return (pltpu.make_async_copy(esrc.at[pl.ds(base + g * K2B, K2B)],
                                      srcb[p], s_idx),
                pltpu.make_async_copy(edst.at[pl.ds(base + g * K2B, K2B)],
                                      dstb[p], s_idx))

    def gather_descs(g, p):
        e0 = base + g * K2B
        return (pltpu.make_async_copy(xl0.at[srcb[p]], bxl0b[p], s_g0),
                pltpu.make_async_copy(xl1.at[srcb[p]], bxl1b[p], s_g1),
                pltpu.make_async_copy(xr0.at[dstb[p]], bxr0b[p], s_g2),
                pltpu.make_async_copy(xr1.at[dstb[p]], bxr1b[p], s_g3),
                pltpu.make_async_copy(ee.at[pl.ds(e0, K2B)], beeb[p], s_ee))

    def log_desc(g, p):
        e0 = base + g * K2B
        return pltpu.make_async_copy(lbufb[p].at[pl.ds(0, K2B)],
                                     logits.at[pl.ds(e0, K2B)],
                                     s_logA if p == 0 else s_logB)

    def compute(g, p, mx16):
        bl0, bl1, br0, br1, be, lb = (bxl0b[p], bxl1b[p], bxr0b[p],
                                      bxr1b[p], beeb[p], lbufb[p])
        for gg in range(3):
            ne = 16 if gg < 2 else K2B - 32

            def edge(je, _):
                j = gg * 16 + je
                t = jnp.zeros((16,), jnp.float32)
                for k in range(8):
                    z = (bl0[j, pl.ds(16 * k, 16)]
                         + br0[j, pl.ds(16 * k, 16)]
                         + be[j, pl.ds(16 * k, 16)])
                    z = jnp.maximum(z, NEG * z)
                    t = t + attc[k] * z
                for k in range(8):
                    z = (bl1[j, pl.ds(16 * k, 16)]
                         + br1[j, pl.ds(16 * k, 16)]
                         + be[j, pl.ds(DH + 16 * k, 16)])
                    z = jnp.maximum(z, NEG * z)
                    t = t + attc[8 + k] * z
                plsc.store_scatter(tbuf, [iota * 16 + je], t)
                return 0

            lax.fori_loop(0, ne, edge, 0, unroll=2)
            colsum = tbuf[pl.ds(0, 16)]
            for r in range(1, 16):
                colsum = colsum + tbuf[pl.ds(16 * r, 16)]
            lb[pl.ds(16 * gg, 16)] = colsum
            mx16 = jnp.maximum(mx16, colsum)
        return mx16

    def body_half(g, p, mx16):
        for d in gather_descs(g, p):
            d.wait()
        for d in idx_descs(g + 1, 1 - p):
            d.wait()
        for d in gather_descs(g + 1, 1 - p):
            d.start()
        gnxt = jnp.minimum(g + 2, NB2 - 1)
        for d in idx_descs(gnxt, p):
            d.start()

        @pl.when(g >= 2)
        def _():
            log_desc(g - 2, p).wait()

        mx16 = compute(g, p, mx16)
        log_desc(g, p).start()
        return mx16

    for d in idx_descs(0, 0):
        d.start()
    for d in idx_descs(0, 0):
        d.wait()
    for d in gather_descs(0, 0):
        d.start()
    for d in idx_descs(1, 1):
        d.start()

    def pair(gp, mx16):
        g = gp * 2
        mx16 = body_half(g, 0, mx16)
        mx16 = body_half(g + 1, 1, mx16)
        return mx16

    mx16 = lax.fori_loop(0, (NB2 - 1) // 2, pair,
                         jnp.full((16,), NEG_INF, jnp.float32))

    gl = NB2 - 1
    for d in gather_descs(gl, 0):
        d.wait()
    log_desc(gl - 2, 0).wait()
    mx16 = compute(gl, 0, mx16)
    log_desc(gl, 0).start()
    for d in idx_descs(gl, 1):
        d.wait()
    log_desc(gl - 1, 1).wait()
    log_desc(gl, 0).wait()

    mxb[...] = mx16
    pltpu.sync_copy(mxb, tmax.at[wid])


def _k2(xl0, xl1, xr0, xr1, ee, esrc, edst, att):
    mesh = plsc.VectorSubcoreMesh(core_axis_name="c", subcore_axis_name="s")
    f = functools.partial(
        pl.kernel,
        out_type=[
            jax.ShapeDtypeStruct((E,), jnp.float32),
            jax.ShapeDtypeStruct((NT2, 16), jnp.float32),
        ],
        mesh=mesh,
        scratch_types=(
            [pltpu.VMEM((K2B,), jnp.int32)] * 4
            + [pltpu.VMEM((K2B, DH), jnp.float32)] * 8
            + [pltpu.VMEM((K2B, D), jnp.float32)] * 2
            + [pltpu.VMEM((48,), jnp.float32)] * 2
            + [pltpu.VMEM((D,), jnp.float32),
               pltpu.VMEM((16,), jnp.float32),
               pltpu.VMEM((D,), jnp.float32)]
            + [pltpu.SemaphoreType.DMA] * 8
        ),
        compiler_params=pltpu.CompilerParams(needs_layout_passes=False),
    )(_k2_body)
    return f(xl0, xl1, xr0, xr1, ee, esrc, edst, att)


def _k25_body(edst, logits, tmax, mself,
              sparts,
              dstb, lb, stab, tb, msb):
    c = lax.axis_index("c")
    s = lax.axis_index("s")
    wid = s * 2 + c
    base = wid * EPT2
    iota = lax.broadcasted_iota(jnp.int32, (16,), 0)

    pltpu.sync_copy(tmax, tb)
    pltpu.sync_copy(mself.at[0, pl.ds(0, 16)], msb)
    m16 = msb[...]
    for r in range(NT2):
        m16 = jnp.maximum(m16, tb[r])
    for sh in (8, 4, 2, 1):
        msb[...] = m16
        m16 = jnp.maximum(m16, plsc.load_gather(msb, [(iota + sh) & 15]))

    z16 = jnp.zeros((16,), jnp.float32)

    def zs(i, _):
        stab[pl.ds(16 * i, 16)] = z16
        return 0

    lax.fori_loop(0, NP // 16, zs, 0)

    pltpu.sync_copy(edst.at[pl.ds(base, EPT2)], dstb)
    pltpu.sync_copy(logits.at[pl.ds(base, EPT2)], lb)

    def grp(gi, _):
        off = gi * 16
        dst16 = dstb[pl.ds(off, 16)]
        ex16 = jnp.exp(lb[pl.ds(off, 16)] - m16)
        for r in range(16):
            plsc.addupdate_scatter(stab, [dst16], ex16, mask=iota == r)
        return 0

    lax.fori_loop(0, EPT2 // 16 , grp, 0)
    dst16 = dstb[pl.ds(EPT2 - 16, 16)]
    ex16 = jnp.exp(lb[pl.ds(EPT2 - 16, 16)] - m16)
    for r in range(8, 16):
        plsc.addupdate_scatter(stab, [dst16], ex16, mask=iota == r)

    pltpu.sync_copy(stab, sparts.at[pl.ds(wid * NP, NP)])


def _k25(edst, logits, tmax, mself):
    mesh = plsc.VectorSubcoreMesh(core_axis_name="c", subcore_axis_name="s")
    f = functools.partial(
        pl.kernel,
        out_type=[jax.ShapeDtypeStruct((NT2 * NP,), jnp.float32)],
        mesh=mesh,
        scratch_types=(
            [pltpu.VMEM((EPT2,), jnp.int32),
             pltpu.VMEM((EPT2,), jnp.float32),
             pltpu.VMEM((NP,), jnp.float32),
             pltpu.VMEM((NT2, 16), jnp.float32),
             pltpu.VMEM((16,), jnp.float32)]
        ),
        compiler_params=pltpu.CompilerParams(needs_layout_passes=False),
    )(_k25_body)
    return f(edst, logits, tmax, mself)[0]


def _k3_body(xl0, xl1, esrc, edst, logits, tmax, mself,
             accout,
             acc, srcbA, srcbB, dstbA, dstbB, dstsA, dstsB,
             gbA, gbB, stageA, stageB,
             lbA, lbB, ebA, ebB, tb, msb,
             s_idx, s_gb, s_lg, s_scA, s_scB):
    c = lax.axis_index("c")
    s = lax.axis_index("s")
    iota = lax.broadcasted_iota(jnp.int32, (16,), 0)
    srcb = (srcbA, srcbB)
    dstb = (dstbA, dstbB)
    dsts = (dstsA, dstsB)
    gbb = (gbA, gbB)
    stage = (stageA, stageB)
    lbb = (lbA, lbB)
    ebb = (ebA, ebB)
    s_sc = (s_scA, s_scB)

    pltpu.sync_copy(tmax, tb)
    pltpu.sync_copy(mself.at[0, pl.ds(0, 16)], msb)
    m16 = msb[...]
    for r in range(NT2):
        m16 = jnp.maximum(m16, tb[r])
    for sh in (8, 4, 2, 1):
        msb[...] = m16
        m16 = jnp.maximum(m16, plsc.load_gather(msb, [(iota + sh) & 15]))

    z16 = jnp.zeros((16,), jnp.float32)

    def zrow(i, _):
        for k in range(DH // 16):
            stageA[i, pl.ds(16 * k, 16)] = z16
        return 0

    lax.fori_loop(0, K3B, zrow, 0)

    def zacc(q, _):
        pltpu.sync_copy(stageA, acc.at[pl.ds(s * 640 + q * K3B, K3B)])
        return 0

    lax.fori_loop(0, 640 // K3B, zacc, 0)
    plsc.subcore_barrier()

    ebase = s * EPT3

    def run(xtab):
        def idx_descs(g, p):
            e0 = ebase + g * K3B
            return (pltpu.make_async_copy(esrc.at[pl.ds(e0, K3B)],
                                          srcb[p], s_idx),
                    pltpu.make_async_copy(edst.at[pl.ds(e0, K3B)],
                                          dstb[p], s_idx))

        def gather_descs(g, p):
            e0 = ebase + g * K3B
            return (pltpu.make_async_copy(xtab.at[srcb[p]], gbb[p], s_gb),
                    pltpu.make_async_copy(logits.at[pl.ds(e0, K3B)],
                                          lbb[p], s_lg))

        def scat_desc(p):
            return pltpu.make_async_copy(stage[p], acc.at[dsts[p]], s_sc[p])

        def scat_start(p):
            pltpu.async_copy(stage[p], acc.at[dsts[p]], s_sc[p], add=True)

        def compute(g, p):
            lb, gb, stg = lbb[p], gbb[p], stage[p]
            exb = ebb[p]
            for off in range(0, K3B, 16):
                dsts[p][pl.ds(off, 16)] = dstb[p][pl.ds(off, 16)]
                exb[pl.ds(off, 16)] = jnp.exp(lb[pl.ds(off, 16)] - m16)

            def edge(j, _):
                exs = plsc.load_gather(exb, [jnp.full((16,), j, jnp.int32)])
                for k in range(DH // 16):
                    stg[j, pl.ds(16 * k, 16)] = gb[j, pl.ds(16 * k, 16)] * exs
                return 0

            lax.fori_loop(0, K3B, edge, 0, unroll=8)

        def body_half(g, p):
            for d in gather_descs(g, p):
                d.wait()
            for d in idx_descs(g + 1, 1 - p):
                d.wait()
            for d in gather_descs(g + 1, 1 - p):
                d.start()

            @pl.when(g >= 2)
            def _():
                scat_desc(p).wait()

            compute(g, p)
            scat_start(p)
            gnxt = jnp.minimum(g + 2, NB3 - 1)
            for d in idx_descs(gnxt, p):
                d.start()

        for d in idx_descs(0, 0):
            d.start()
        for d in idx_descs(0, 0):
            d.wait()
        for d in gather_descs(0, 0):
            d.start()
        for d in idx_descs(1, 1):
            d.start()

        def pair(gp, _):
            g = gp * 2
            body_half(g, 0)
            body_half(g + 1, 1)
            return 0

        lax.fori_loop(0, (NB3 - 1) // 2, pair, 0)

        gl = NB3 - 1
        for d in gather_descs(gl, 0):
            d.wait()
        scat_desc(0).wait()
        compute(jnp.int32(gl), 0)
        scat_start(0)
        for d in idx_descs(gl, 1):
            d.wait()
        scat_desc(1).wait()
        scat_desc(0).wait()

    @pl.when(c == 0)
    def _():
        run(xl0)

    @pl.when(c == 1)
    def _():
        run(xl1)

    plsc.subcore_barrier()
    pltpu.sync_copy(acc.at[pl.ds(s * 640, 640)],
                    accout.at[pl.ds(c * NP + s * 640, 640)])


def _k3(xl0, xl1, esrc, edst, logits, tmax, mself):
    mesh = plsc.VectorSubcoreMesh(core_axis_name="c", subcore_axis_name="s")
    f = functools.partial(
        pl.kernel,
        out_type=[
            jax.ShapeDtypeStruct((2 * NP, DH), jnp.float32),
        ],
        mesh=mesh,
        scratch_types=(
            [pltpu.VMEM_SHARED((NP, DH), jnp.float32)]
            + [pltpu.VMEM((K3B,), jnp.int32)] * 6
            + [pltpu.VMEM((K3B, DH), jnp.float32)] * 4
            + [pltpu.VMEM((K3B,), jnp.float32)] * 4
            + [pltpu.VMEM((NT2, 16), jnp.float32),
               pltpu.VMEM((16,), jnp.float32)]
            + [pltpu.SemaphoreType.DMA] * 5
        ),
        compiler_params=pltpu.CompilerParams(needs_layout_passes=False),
    )(_k3_body)
    return f(xl0, xl1, esrc, edst, logits, tmax, mself)[0]


def _k4_body(a0_ref, a1_ref, sp_ref, xl0_ref, xl1_ref, xr0_ref, xr1_ref,
             tmax_ref, m_ref, sum_ref, we_ref, att_ref, bias_ref, out_ref):
    M = jnp.maximum(jnp.max(tmax_ref[...]), jnp.max(m_ref[...]))
    crow = jnp.dot(sum_ref[...] * (1.0 / E), we_ref[...],
                   preferred_element_type=jnp.float32)
    xl = jnp.concatenate([xl0_ref[...], xl1_ref[...]], axis=1)
    xr = jnp.concatenate([xr0_ref[...], xr1_ref[...]], axis=1)
    z = xl + xr + crow
    z = jnp.maximum(z, NEG * z)
    ls = jnp.sum(z * att_ref[...], axis=1, keepdims=True)
    exs = jnp.exp(ls - M)
    ssum = jnp.sum(sp_ref[...], axis=0)[:, None] + exs
    num = jnp.concatenate([a0_ref[...], a1_ref[...]], axis=1) + exs * xl
    out_ref[...] = jnp.maximum(num / ssum + bias_ref[...], 0.0)


def _k4(accout, sparts2, xl0, xl1, xr0, xr1, tmax, mself, sumea, W_e,
        att2, bias2):
    return pl.pallas_call(
        _k4_body,
        grid=(NP // NBLK,),
        in_specs=[
            pl.BlockSpec((NBLK, DH), lambda i: (i, 0)),
            pl.BlockSpec((NBLK, DH), lambda i: (i + NP // NBLK, 0)),
            pl.BlockSpec((NT2, NBLK), lambda i: (0, i)),
            pl.BlockSpec((NBLK, DH), lambda i: (i, 0)),
            pl.BlockSpec((NBLK, DH), lambda i: (i, 0)),
            pl.BlockSpec((NBLK, DH), lambda i: (i, 0)),
            pl.BlockSpec((NBLK, DH), lambda i: (i, 0)),
            pl.BlockSpec((NT2, 16), lambda i: (0, 0)),
            pl.BlockSpec((8, 128), lambda i: (0, 0)),
            pl.BlockSpec((1, DE), lambda i: (0, 0)),
            pl.BlockSpec((DE, D), lambda i: (0, 0)),
            pl.BlockSpec((1, D), lambda i: (0, 0)),
            pl.BlockSpec((1, D), lambda i: (0, 0)),
        ],
        out_specs=pl.BlockSpec((NBLK, D), lambda i: (i, 0)),
        out_shape=jax.ShapeDtypeStruct((NP, D), jnp.float32),
    )(accout, accout, sparts2, xl0, xl1, xr0, xr1, tmax, mself, sumea, W_e,
      att2, bias2)


def kernel(x, edge_index, edge_attr, return_attention_weights,
           W_l, W_r, W_e, att, bias):
    ei = edge_index.astype(jnp.int32)
    esrc = ei[0]
    edst = ei[1]
    att2 = att.reshape(1, D)
    bias2 = bias.reshape(1, D)
    xp = jnp.zeros((NP, D), jnp.float32).at[:N].set(x)
    ee, sumea = _kee(edge_attr, W_e)
    xl0, xl1, xr0, xr1, mself = _k1(xp, W_l, W_r, W_e, sumea, att2)
    logits, tmax = _k2(xl0, xl1, xr0, xr1, ee, esrc, edst, att)
    sparts = _k25(edst, logits, tmax, mself)
    accout = _k3(xl0, xl1, esrc, edst, logits, tmax, mself)
    sparts2 = sparts.reshape(NT2, NP)
    out = _k4(accout, sparts2, xl0, xl1, xr0, xr1, tmax, mself, sumea, W_e,
              att2, bias2)
    return out[:N]
